# baseline (speedup 1.0000x reference)
import functools

import jax
import jax.numpy as jnp
from jax import lax
from jax.experimental import pallas as pl
from jax.experimental.pallas import tpu as pltpu
from jax.experimental.pallas import tpu_sc as plsc

N = 50000
E = 800000
H = 64
B = 64
OUT_C = 51

NC = 2
NS = 16
NW = NC * NS
L = 16

NPAD = 50048
SLICE = NPAD // NS
G = E // L

GRP_LO = G // NW
EXTRA = G - GRP_LO * NW

_mesh = plsc.VectorSubcoreMesh(core_axis_name="c", subcore_axis_name="s")
_sc_params = pltpu.CompilerParams(needs_layout_passes=False,
                                  use_tc_tiling_on_sc=False)


def _wid(c, s):
    return s * NC + c


def _edge_span_32way(w):
    base = w * GRP_LO + jnp.minimum(w, EXTRA)
    ngrp = GRP_LO + jnp.where(w < EXTRA, 1, 0)
    return base * L, ngrp


def _k1_body(x_hbm, src_hbm, dst_hbm, w_hbm, out_hbm,
             x_v,
             srcb0, dstb0, wb0, updb0, dsts0,
             srcb1, dstb1, wb1, updb1, dsts1,
             srcb16, dstb16, wb16, updb16,
             zbuf, si0, si1, ss0, ss1, acc_sh):
    c = lax.axis_index("c")
    s = lax.axis_index("s")
    w = _wid(c, s)
    bufs = ((srcb0, dstb0, wb0, updb0, dsts0, si0, ss0),
            (srcb1, dstb1, wb1, updb1, dsts1, si1, ss1))

    def zstore(i, _):
        zbuf[pl.ds(i * 16, 16)] = jnp.zeros((16,), jnp.float32)
        return 0

    lax.fori_loop(0, 200, zstore, 0)
    pltpu.sync_copy(zbuf.at[pl.ds(0, SLICE)], acc_sh.at[pl.ds(s * SLICE, SLICE)])
    pltpu.sync_copy(x_hbm, x_v)
    plsc.subcore_barrier()

    ebase, ngrp = _edge_span_32way(w)
    nch = GRP_LO // 8
    z16 = jnp.zeros((16,), jnp.float32)
    zi16 = jnp.zeros((16,), jnp.int32)

    def fire_in(k, t):
        srcb, dstb, wb, _, _, si, _ = bufs[t]
        b = ebase + k * 128
        pltpu.async_copy(src_hbm.at[pl.ds(b, 128)], srcb, si)
        pltpu.async_copy(dst_hbm.at[pl.ds(b, 128)], dstb.at[0], si)
        pltpu.async_copy(w_hbm.at[pl.ds(b, 128)], wb, si)

    def wait_in(t):
        srcb, dstb, wb, _, _, si, _ = bufs[t]
        pltpu.make_async_copy(src_hbm.at[pl.ds(0, 128)], srcb, si).wait()
        pltpu.make_async_copy(dst_hbm.at[pl.ds(0, 128)], dstb.at[0], si).wait()
        pltpu.make_async_copy(w_hbm.at[pl.ds(0, 128)], wb, si).wait()

    for t in range(2):
        srcb, dstb, wb, updb, dsts, si, ss = bufs[t]
        for g in range(8):
            updb[pl.ds(g * 16, 16)] = z16
            dsts[0, pl.ds(g * 16, 16)] = zi16
        pltpu.async_copy(updb, acc_sh.at[dsts.at[0]], ss, add=True)
        fire_in(t, t)

    def body(i, _):
        for t in range(2):
            k = 2 * i + t
            srcb, dstb, wb, updb, dsts, si, ss = bufs[t]
            wait_in(t)
            pltpu.make_async_copy(updb, acc_sh.at[dsts.at[0]], ss).wait()
            for g in range(8):
                sl = pl.ds(g * 16, 16)
                xg = plsc.load_gather(x_v, [srcb[sl]])
                updb[sl] = xg * wb[sl]
                dsts[0, sl] = dstb[0, sl]
            pltpu.async_copy(updb, acc_sh.at[dsts.at[0]], ss, add=True)

            @pl.when(k + 2 < nch - 1)
            def _():
                fire_in(k + 2, t)

        return 0

    lax.fori_loop(0, nch // 2, body, 0)
    pltpu.make_async_copy(updb0, acc_sh.at[dsts0.at[0]], ss0).wait()
    pltpu.make_async_copy(updb1, acc_sh.at[dsts1.at[0]], ss1).wait()

    def rem(k, _):
        b = ebase + (nch - 1) * 128 + k * 16
        pltpu.sync_copy(src_hbm.at[pl.ds(b, 16)], srcb16)
        pltpu.sync_copy(dst_hbm.at[pl.ds(b, 16)], dstb16.at[0])
        pltpu.sync_copy(w_hbm.at[pl.ds(b, 16)], wb16)
        xg = plsc.load_gather(x_v, [srcb16[...]])
        updb16[...] = xg * wb16[...]
        pltpu.sync_copy(updb16, acc_sh.at[dstb16.at[0]], add=True)
        return 0

    lax.fori_loop(0, 8 + ngrp - nch * 8, rem, 0)

    plsc.subcore_barrier()
    pltpu.sync_copy(acc_sh.at[pl.ds(s * SLICE, SLICE)], zbuf.at[pl.ds(0, SLICE)])
    pltpu.sync_copy(zbuf.at[pl.ds(0, SLICE)],
                    out_hbm.at[pl.ds(c * NPAD + s * SLICE, SLICE)])


@functools.partial(
    pl.kernel,
    out_type=jax.ShapeDtypeStruct((NC * NPAD,), jnp.float32),
    mesh=_mesh,
    scratch_types=(
        [pltpu.VMEM((N,), jnp.float32)]
        + [
            pltpu.VMEM((128,), jnp.int32),
            pltpu.VMEM((1, 128), jnp.int32),
            pltpu.VMEM((128,), jnp.float32),
            pltpu.VMEM((128,), jnp.float32),
            pltpu.VMEM((1, 128), jnp.int32),
        ] * 2
        + [
            pltpu.VMEM((16,), jnp.int32),
            pltpu.VMEM((1, 16), jnp.int32),
            pltpu.VMEM((16,), jnp.float32),
            pltpu.VMEM((16,), jnp.float32),
            pltpu.VMEM((3200,), jnp.float32),
        ]
        + [pltpu.SemaphoreType.DMA] * 4
        + [pltpu.VMEM_SHARED((NPAD,), jnp.float32)]
    ),
    compiler_params=_sc_params,
)
def _k1(x_hbm, src_hbm, dst_hbm, w_hbm, out_hbm, *scratch):
    _k1_body(x_hbm, src_hbm, dst_hbm, w_hbm, out_hbm, *scratch)


K2BLK = 2000


def _k2_body(aP_ref, x_ref, u_ref, v_ref, b1_ref, h1full_ref, asum_ref):
    a = aP_ref[0] + aP_ref[1]
    asum_ref[...] = a
    h = a * u_ref[...] + x_ref[...] * v_ref[...] + b1_ref[...]
    h1full_ref[...] = jnp.maximum(h, 0.0)


def _k2(aP3, x, u, v, b1r):
    grid = N // K2BLK
    return pl.pallas_call(
        _k2_body,
        grid=(grid,),
        in_specs=[
            pl.BlockSpec((NC, K2BLK, 1), lambda i: (0, i, 0)),
            pl.BlockSpec((K2BLK, 1), lambda i: (i, 0)),
            pl.BlockSpec((1, H), lambda i: (0, 0)),
            pl.BlockSpec((1, H), lambda i: (0, 0)),
            pl.BlockSpec((1, H), lambda i: (0, 0)),
        ],
        out_specs=[
            pl.BlockSpec((K2BLK, H), lambda i: (i, 0)),
            pl.BlockSpec((K2BLK, 1), lambda i: (i, 0)),
        ],
        out_shape=[
            jax.ShapeDtypeStruct((N, H), jnp.float32),
            jax.ShapeDtypeStruct((N, 1), jnp.float32),
        ],
    )(aP3, x, u, v, b1r)


GPT = G // NS
K3CH = GPT // 8
K3REM = GPT - K3CH * 8

_K3PIECES = tuple((k * 200, 200) for k in range(15)) + ((3000, 128),)


def _k3_body(a_hbm, x_hbm, u_hbm, v_hbm, b_hbm, src_hbm, dst_hbm, w_hbm,
             out_hbm,
             bnc, u_v, v_v, b_v, u_sm, v_sm, b_sm,
             srcb0, dstb0, wb0, abuf0, xbuf0, upd0, dsts0,
             srcb1, dstb1, wb1, abuf1, xbuf1, upd1, dsts1,
             dstb80, zbuf, si0, si1, sg0, sg1, ss0, ss1,
             uvb_sp, a_sp, x_sp, acc_sh):
    c = lax.axis_index("c")
    s = lax.axis_index("s")
    bufs = ((srcb0, dstb0, wb0, abuf0, xbuf0, upd0, dsts0, si0, sg0, ss0),
            (srcb1, dstb1, wb1, abuf1, xbuf1, upd1, dsts1, si1, sg1, ss1))

    def zrow(r, _):
        zbuf[r, pl.ds(0, 16)] = jnp.zeros((16,), jnp.float32)
        zbuf[r, pl.ds(16, 16)] = jnp.zeros((16,), jnp.float32)
        return 0

    lax.fori_loop(0, 200, zrow, 0)
    for off, ln in _K3PIECES:
        pltpu.sync_copy(zbuf.at[pl.ds(0, ln), :],
                        acc_sh.at[pl.ds(s * SLICE + off, ln), :])
    sl_me = pl.ds(s * SLICE, SLICE)
    bsl = pl.ds(0, SLICE)
    pltpu.sync_copy(a_hbm.at[sl_me], bnc.at[bsl])
    pltpu.sync_copy(bnc.at[bsl], a_sp.at[sl_me])
    pltpu.sync_copy(x_hbm.at[sl_me], bnc.at[bsl])
    pltpu.sync_copy(bnc.at[bsl], x_sp.at[sl_me])
    pltpu.sync_copy(u_hbm, u_v)
    pltpu.sync_copy(v_hbm, v_v)
    pltpu.sync_copy(b_hbm, b_v)
    pltpu.sync_copy(u_v, uvb_sp.at[0])
    pltpu.sync_copy(v_v, uvb_sp.at[1])
    pltpu.sync_copy(b_v, uvb_sp.at[2])
    pltpu.sync_copy(uvb_sp.at[0], u_sm)
    pltpu.sync_copy(uvb_sp.at[1], v_sm)
    pltpu.sync_copy(uvb_sp.at[2], b_sm)
    plsc.subcore_barrier()

    coff = c * 32
    ebase = s * (GPT * L)
    z16 = jnp.zeros((16,), jnp.float32)
    zi16 = jnp.zeros((16,), jnp.int32)
    iota16 = lax.iota(jnp.int32, 16)

    def fire_in(k, t):
        srcb, dstb, wb = bufs[t][0], bufs[t][1], bufs[t][2]
        si = bufs[t][7]
        b = ebase + k * 128
        pltpu.async_copy(src_hbm.at[pl.ds(b, 128)], srcb, si)
        pltpu.async_copy(dst_hbm.at[pl.ds(b, 128)], dstb.at[0], si)
        pltpu.async_copy(w_hbm.at[pl.ds(b, 128)], wb, si)

    def wait_in(t):
        srcb, dstb, wb = bufs[t][0], bufs[t][1], bufs[t][2]
        si = bufs[t][7]
        pltpu.make_async_copy(src_hbm.at[pl.ds(0, 128)], srcb, si).wait()
        pltpu.make_async_copy(dst_hbm.at[pl.ds(0, 128)], dstb.at[0], si).wait()
        pltpu.make_async_copy(w_hbm.at[pl.ds(0, 128)], wb, si).wait()

    def compute(abuf, xbuf, wb, upd, nedges):
        def grp(g, _):
            e0 = g * 16
            er = pl.ds(e0, 16)
            a16 = abuf[er]
            x16 = xbuf[er]
            w16 = wb[er]
            rowi = e0 + iota16
            for cc in range(32):
                us = u_sm[coff + cc]
                vs = v_sm[coff + cc]
                bs = b_sm[coff + cc]
                val = jnp.maximum(a16 * us + x16 * vs + bs, 0.0) * w16
                plsc.store_scatter(upd, [rowi, jnp.full((16,), cc, jnp.int32)],
                                   val)
            return 0

        lax.fori_loop(0, nedges // 16, grp, 0)

    def fire_gathers(t):
        srcb, abuf, xbuf = bufs[t][0], bufs[t][3], bufs[t][4]
        sg = bufs[t][8]
        pltpu.async_copy(a_sp.at[srcb], abuf, sg)
        pltpu.async_copy(x_sp.at[srcb], xbuf, sg)

    def wait_gathers(t):
        srcb, abuf, xbuf = bufs[t][0], bufs[t][3], bufs[t][4]
        sg = bufs[t][8]
        pltpu.make_async_copy(a_sp.at[srcb], abuf, sg).wait()
        pltpu.make_async_copy(x_sp.at[srcb], xbuf, sg).wait()

    for t in range(2):
        srcb, dstb, wb, abuf, xbuf, upd, dsts, si, sg, ss = bufs[t]

        def zupd(r, _, upd=upd):
            upd[r, pl.ds(0, 16)] = z16
            upd[r, pl.ds(16, 16)] = z16
            return 0

        lax.fori_loop(0, 128, zupd, 0)
        for g in range(8):
            dsts[0, pl.ds(g * 16, 16)] = zi16
        pltpu.async_copy(upd, acc_sh.at[dsts.at[0]], ss, add=True)
        fire_in(t, t)
    wait_in(0)
    fire_gathers(0)
    wait_in(1)
    fire_gathers(1)

    def body(i, _):
        for t in range(2):
            k = 2 * i + t
            srcb, dstb, wb, abuf, xbuf, upd, dsts, si, sg, ss = bufs[t]
            pltpu.make_async_copy(upd, acc_sh.at[dsts.at[0]], ss).wait()
            for g in range(8):
                dsts[0, pl.ds(g * 16, 16)] = dstb[0, pl.ds(g * 16, 16)]
            wait_gathers(t)
            compute(abuf, xbuf, wb, upd, 128)
            pltpu.async_copy(upd, acc_sh.at[dsts.at[0]], ss, add=True)

            @pl.when(k + 2 < K3CH)
            def _():
                fire_in(k + 2, t)

        @pl.when(2 * i + 2 < K3CH)
        def _():
            wait_in(0)
            fire_gathers(0)

        @pl.when(2 * i + 3 < K3CH)
        def _():
            wait_in(1)
            fire_gathers(1)

        return 0

    lax.fori_loop(0, K3CH // 2, body, 0)
    pltpu.make_async_copy(upd0, acc_sh.at[dsts0.at[0]], ss0).wait()
    pltpu.make_async_copy(upd1, acc_sh.at[dsts1.at[0]], ss1).wait()

    b = ebase + K3CH * 128
    pltpu.sync_copy(src_hbm.at[pl.ds(b, 80)], srcb0.at[pl.ds(0, 80)])
    pltpu.sync_copy(dst_hbm.at[pl.ds(b, 80)], dstb80.at[0])
    pltpu.sync_copy(w_hbm.at[pl.ds(b, 80)], wb0.at[pl.ds(0, 80)])
    pltpu.async_copy(a_sp.at[srcb0.at[pl.ds(0, 80)]], abuf0.at[pl.ds(0, 80)],
                     sg0).wait()
    pltpu.async_copy(x_sp.at[srcb0.at[pl.ds(0, 80)]], xbuf0.at[pl.ds(0, 80)],
                     sg0).wait()
    compute(abuf0, xbuf0, wb0, upd0, 80)
    pltpu.sync_copy(upd0.at[pl.ds(0, 80), :], acc_sh.at[dstb80.at[0]], add=True)

    plsc.subcore_barrier()
    for off, ln in _K3PIECES:
        pltpu.sync_copy(acc_sh.at[pl.ds(s * SLICE + off, ln), :],
                        zbuf.at[pl.ds(0, ln), :])
        pltpu.sync_copy(zbuf.at[pl.ds(0, ln), :],
                        out_hbm.at[c, pl.ds(s * SLICE + off, ln), :])


@functools.partial(
    pl.kernel,
    out_type=jax.ShapeDtypeStruct((NC, NPAD, 32), jnp.float32),
    mesh=_mesh,
    scratch_types=(
        [
            pltpu.VMEM((SLICE,), jnp.float32),
            pltpu.VMEM((H,), jnp.float32),
            pltpu.VMEM((H,), jnp.float32),
            pltpu.VMEM((H,), jnp.float32),
            pltpu.SMEM((H,), jnp.float32),
            pltpu.SMEM((H,), jnp.float32),
            pltpu.SMEM((H,), jnp.float32),
        ]
        + [
            pltpu.VMEM((128,), jnp.int32),
            pltpu.VMEM((1, 128), jnp.int32),
            pltpu.VMEM((128,), jnp.float32),
            pltpu.VMEM((128,), jnp.float32),
            pltpu.VMEM((128,), jnp.float32),
            pltpu.VMEM((128, 32), jnp.float32),
            pltpu.VMEM((1, 128), jnp.int32),
        ] * 2
        + [
            pltpu.VMEM((1, 80), jnp.int32),
            pltpu.VMEM((200, 32), jnp.float32),
        ]
        + [pltpu.SemaphoreType.DMA] * 6
        + [
            pltpu.VMEM_SHARED((3, H), jnp.float32),
            pltpu.VMEM_SHARED((NPAD,), jnp.float32),
            pltpu.VMEM_SHARED((NPAD,), jnp.float32),
            pltpu.VMEM_SHARED((NPAD, 32), jnp.float32),
        ]
    ),
    compiler_params=_sc_params,
)
def _k3(a_hbm, x_hbm, u_hbm, v_hbm, b_hbm, src_hbm, dst_hbm, w_hbm, out_hbm,
        *scratch):
    _k3_body(a_hbm, x_hbm, u_hbm, v_hbm, b_hbm, src_hbm, dst_hbm, w_hbm,
             out_hbm, *scratch)


def _k4_body(agg_ref, h1_ref, Wrel_ref, b2_ref, Wroot_ref, h2_ref):
    a0 = agg_ref[0]
    a1 = agg_ref[1]
    Wr = Wrel_ref[...]
    dn = (((1,), (1,)), ((), ()))
    h = (lax.dot_general(a0, Wr[:, :32], dn)
         + lax.dot_general(a1, Wr[:, 32:], dn)
         + lax.dot_general(h1_ref[...], Wroot_ref[...], dn)
         + b2_ref[...])
    h2_ref[...] = jnp.maximum(h, 0.0)


def _k4(aggcat, h1full, W2_rel, b2r, W2_root):
    grid = N // K2BLK
    return pl.pallas_call(
        _k4_body,
        grid=(grid,),
        in_specs=[
            pl.BlockSpec((NC, K2BLK, 32), lambda i: (0, i, 0)),
            pl.BlockSpec((K2BLK, H), lambda i: (i, 0)),
            pl.BlockSpec((H, H), lambda i: (0, 0)),
            pl.BlockSpec((1, H), lambda i: (0, 0)),
            pl.BlockSpec((H, H), lambda i: (0, 0)),
        ],
        out_specs=pl.BlockSpec((K2BLK, H), lambda i: (i, 0)),
        out_shape=jax.ShapeDtypeStruct((N, H), jnp.float32),
    )(aggcat, h1full, W2_rel, b2r, W2_root)


def _k5_body(h2p_hbm, batch_hbm, src_hbm, dst_hbm, w_hbm, out_hbm,
             batch_v,
             srcb0, gib0, pbuf0, dstb0, wb0, rows0,
             srcb1, gib1, pbuf1, dstb1, wb1, rows1,
             gbuf, srcb16, gib16, pbuf16, dstb16, wb16, rows16,
             si0, si1, sg0, sg1, acc):
    c = lax.axis_index("c")
    s = lax.axis_index("s")
    w = _wid(c, s)
    bufs = ((srcb0, gib0, pbuf0, dstb0, wb0, rows0, si0, sg0),
            (srcb1, gib1, pbuf1, dstb1, wb1, rows1, si1, sg1))

    pltpu.sync_copy(batch_hbm, batch_v)

    def zrow(r, _):
        for c0 in range(4):
            acc[r, pl.ds(c0 * 16, 16)] = jnp.zeros((16,), jnp.float32)
        return 0

    lax.fori_loop(0, B, zrow, 0)

    ebase, ngrp = _edge_span_32way(w)
    nch = GRP_LO // 8
    iota = lax.iota(jnp.int32, 16)

    def fire_in(k, t):
        srcb, _, _, dstb, wb = bufs[t][0], None, None, bufs[t][3], bufs[t][4]
        si = bufs[t][6]
        b = ebase + k * 128
        pltpu.async_copy(src_hbm.at[pl.ds(b, 128)], srcb, si)
        pltpu.async_copy(dst_hbm.at[pl.ds(b, 128)], dstb, si)
        pltpu.async_copy(w_hbm.at[pl.ds(b, 128)], wb, si)

    def wait_in(t):
        srcb, dstb, wb = bufs[t][0], bufs[t][3], bufs[t][4]
        si = bufs[t][6]
        pltpu.make_async_copy(src_hbm.at[pl.ds(0, 128)], srcb, si).wait()
        pltpu.make_async_copy(dst_hbm.at[pl.ds(0, 128)], dstb, si).wait()
        pltpu.make_async_copy(w_hbm.at[pl.ds(0, 128)], wb, si).wait()

    def prep(t):
        srcb, gib, pbuf, rows = bufs[t][0], bufs[t][1], bufs[t][2], bufs[t][5]
        sg = bufs[t][7]
        for g in range(8):
            sl = pl.ds(g * 16, 16)
            si = srcb[sl]
            gib[sl] = lax.shift_right_logical(si, 1)
            pbuf[sl] = (si & 1) * 64
        pltpu.async_copy(h2p_hbm.at[gib], rows, sg)

    def accumulate(t):
        gib, pref, dref, wref, rows = (bufs[t][1], bufs[t][2], bufs[t][3],
                                       bufs[t][4], bufs[t][5])
        sg = bufs[t][7]
        pltpu.make_async_copy(h2p_hbm.at[gib], rows, sg).wait()
        for g in range(8):
            sl = pl.ds(g * 16, 16)
            gbuf[sl] = plsc.load_gather(batch_v, [dref[sl]])

        def acc8(m, _):
            for tt in range(8):
                e = m * 8 + tt
                esp = jnp.full((16,), e, jnp.int32)
                wsp = plsc.load_gather(wref, [esp])
                gsp = plsc.load_gather(gbuf, [esp])
                psp = plsc.load_gather(pref, [esp])
                for c0 in range(4):
                    v = plsc.load_gather(rows, [esp, psp + (iota + c0 * 16)])
                    plsc.addupdate_scatter(acc, [gsp, iota + c0 * 16], v * wsp)
            return 0

        lax.fori_loop(0, 16, acc8, 0)

    fire_in(0, 0)
    fire_in(1, 1)
    wait_in(0)
    prep(0)

    def body(i, _):
        wait_in(1)
        prep(1)
        accumulate(0)

        @pl.when(2 * i + 2 < nch)
        def _():
            fire_in(2 * i + 2, 0)

        accumulate(1)

        @pl.when(2 * i + 3 < nch)
        def _():
            fire_in(2 * i + 3, 1)

        @pl.when(2 * i + 2 < nch)
        def _():
            wait_in(0)
            prep(0)

        return 0

    lax.fori_loop(0, nch // 2, body, 0)
    accumulate(0)

    def rem(k, _):
        b = ebase + nch * 128 + k * 16
        pltpu.sync_copy(src_hbm.at[pl.ds(b, 16)], srcb16)
        pltpu.sync_copy(dst_hbm.at[pl.ds(b, 16)], dstb16)
        pltpu.sync_copy(w_hbm.at[pl.ds(b, 16)], wb16)
        si = srcb16[...]
        gib16[...] = lax.shift_right_logical(si, 1)
        pbuf16[...] = (si & 1) * 64
        pltpu.async_copy(h2p_hbm.at[gib16], rows16, sg0).wait()
        for g in range(1):
            gbuf[pl.ds(0, 16)] = plsc.load_gather(batch_v, [dstb16[...]])

        def acc1(m, _):
            for tt in range(8):
                e = m * 8 + tt
                esp = jnp.full((16,), e, jnp.int32)
                wsp = plsc.load_gather(wb16, [esp])
                gsp = plsc.load_gather(gbuf, [esp])
                psp = plsc.load_gather(pbuf16, [esp])
                for c0 in range(4):
                    v = plsc.load_gather(rows16, [esp, psp + (iota + c0 * 16)])
                    plsc.addupdate_scatter(acc, [gsp, iota + c0 * 16], v * wsp)
            return 0

        lax.fori_loop(0, 2, acc1, 0)
        return 0

    lax.fori_loop(0, ngrp - nch * 8, rem, 0)

    pltpu.sync_copy(acc, out_hbm.at[w])


@functools.partial(
    pl.kernel,
    out_type=jax.ShapeDtypeStruct((NW, B, H), jnp.float32),
    mesh=_mesh,
    scratch_types=(
        [pltpu.VMEM((N,), jnp.int32)]
        + [
            pltpu.VMEM((128,), jnp.int32),
            pltpu.VMEM((128,), jnp.int32),
            pltpu.VMEM((128,), jnp.int32),
            pltpu.VMEM((128,), jnp.int32),
            pltpu.VMEM((128,), jnp.float32),
            pltpu.VMEM((128, 128), jnp.float32),
        ] * 2
        + [
            pltpu.VMEM((128,), jnp.int32),
            pltpu.VMEM((16,), jnp.int32),
            pltpu.VMEM((16,), jnp.int32),
            pltpu.VMEM((16,), jnp.int32),
            pltpu.VMEM((16,), jnp.int32),
            pltpu.VMEM((16,), jnp.float32),
            pltpu.VMEM((16, 128), jnp.float32),
        ]
        + [pltpu.SemaphoreType.DMA] * 4
        + [pltpu.VMEM((B, H), jnp.float32)]
    ),
    compiler_params=_sc_params,
)
def _k5(h2p_hbm, batch_hbm, src_hbm, dst_hbm, w_hbm, out_hbm, *scratch):
    _k5_body(h2p_hbm, batch_hbm, src_hbm, dst_hbm, w_hbm, out_hbm, *scratch)


K6BLK = 2000


def _k6_body(h2_ref, batch_ref, parts_ref, Wrel_ref, b3_ref, Wroot_ref,
             Wlin_ref, blin_ref, out_ref, accP_ref, cnt_ref):
    i = pl.program_id(0)

    @pl.when(i == 0)
    def _():
        accP_ref[...] = jnp.zeros((B, H), jnp.float32)
        cnt_ref[...] = jnp.zeros((B, 1), jnp.float32)

    bb = batch_ref[0]
    oneT = (jnp.broadcast_to(bb, (B, K6BLK))
            == lax.broadcasted_iota(jnp.int32, (B, K6BLK), 0)
            ).astype(jnp.float32)
    dn_rowsum = (((1,), (0,)), ((), ()))
    accP_ref[...] += lax.dot_general(oneT, h2_ref[...], dn_rowsum)
    cnt_ref[...] += jnp.sum(oneT, axis=1, keepdims=True)

    @pl.when(i == (N // K6BLK) - 1)
    def _():
        Pagg = jnp.sum(parts_ref[...], axis=0)
        cnt = cnt_ref[...]
        cm = jnp.maximum(cnt, 1.0)
        dn = (((1,), (1,)), ((), ()))
        pooled = (lax.dot_general(Pagg, Wrel_ref[...], dn)
                  + cnt * b3_ref[...]
                  + lax.dot_general(accP_ref[...], Wroot_ref[...], dn)) / cm
        out_ref[...] = lax.dot_general(pooled, Wlin_ref[...], dn) + blin_ref[...]


def _k6(h2full, batch3, parts, W3_rel, b3r, W3_root, Wlin, blinr):
    grid = N // K6BLK
    return pl.pallas_call(
        _k6_body,
        grid=(grid,),
        in_specs=[
            pl.BlockSpec((K6BLK, H), lambda i: (i, 0)),
            pl.BlockSpec((1, 1, K6BLK), lambda i: (i, 0, 0)),
            pl.BlockSpec((NW, B, H), lambda i: (0, 0, 0)),
            pl.BlockSpec((H, H), lambda i: (0, 0)),
            pl.BlockSpec((1, H), lambda i: (0, 0)),
            pl.BlockSpec((H, H), lambda i: (0, 0)),
            pl.BlockSpec((OUT_C, H), lambda i: (0, 0)),
            pl.BlockSpec((1, OUT_C), lambda i: (0, 0)),
        ],
        out_specs=pl.BlockSpec((B, OUT_C), lambda i: (0, 0)),
        out_shape=jax.ShapeDtypeStruct((B, OUT_C), jnp.float32),
        scratch_shapes=[
            pltpu.VMEM((B, H), jnp.float32),
            pltpu.VMEM((B, 1), jnp.float32),
        ],
    )(h2full, batch3, parts, W3_rel, b3r, W3_root, Wlin, blinr)


def kernel(x, edge_index, batch, edge_weight,
           W1_rel, b1, W1_root, W2_rel, b2, W2_root,
           W3_rel, b3, W3_root, Wlin, blin):
    x1d = x[:, 0]
    src = edge_index[0]
    dst = edge_index[1]

    aP = _k1(x1d, src, dst, edge_weight)
    aP3 = aP.reshape(NC, NPAD, 1)

    u = W1_rel.reshape(1, H)
    v = W1_root.reshape(1, H)
    h1full, asum = _k2(aP3, x, u, v, b1.reshape(1, H))

    apad = jnp.pad(asum.reshape(N), (0, NPAD - N))
    xpad = jnp.pad(x1d, (0, NPAD - N))
    agg2cat = _k3(apad, xpad, W1_rel.reshape(H),
                  W1_root.reshape(H), b1, src, dst, edge_weight)
    h2full = _k4(agg2cat, h1full, W2_rel, b2.reshape(1, H), W2_root)

    h2pack = h2full.reshape(N // 2, 2 * H)
    parts = _k5(h2pack, batch, src, dst, edge_weight)

    batch3 = batch.reshape(N // K6BLK, 1, K6BLK)
    out = _k6(h2full, batch3, parts, W3_rel, b3.reshape(1, H),
              W3_root, Wlin, blin.reshape(1, OUT_C))
    return out


# speedup vs baseline: 1.2504x; 1.2504x over previous
import functools

import jax
import jax.numpy as jnp
from jax import lax
from jax.experimental import pallas as pl
from jax.experimental.pallas import tpu as pltpu
from jax.experimental.pallas import tpu_sc as plsc

N = 50000
E = 800000
H = 64
B = 64
OUT_C = 51

NC = 2
NS = 16
NW = NC * NS
L = 16

NPAD = 50048
SLICE = NPAD // NS
G = E // L

GRP_LO = G // NW
EXTRA = G - GRP_LO * NW

_mesh = plsc.VectorSubcoreMesh(core_axis_name="c", subcore_axis_name="s")
_sc_params = pltpu.CompilerParams(needs_layout_passes=False,
                                  use_tc_tiling_on_sc=False)


def _wid(c, s):
    return s * NC + c


def _edge_span_32way(w):
    base = w * GRP_LO + jnp.minimum(w, EXTRA)
    ngrp = GRP_LO + jnp.where(w < EXTRA, 1, 0)
    return base * L, ngrp


def _k1_body(x_hbm, src_hbm, dst_hbm, w_hbm, out_hbm,
             x_v,
             srcb0, dstb0, wb0, updb0, dsts0,
             srcb1, dstb1, wb1, updb1, dsts1,
             srcb16, dstb16, wb16, updb16,
             zbuf, si0, si1, ss0, ss1, acc_sh):
    c = lax.axis_index("c")
    s = lax.axis_index("s")
    w = _wid(c, s)
    bufs = ((srcb0, dstb0, wb0, updb0, dsts0, si0, ss0),
            (srcb1, dstb1, wb1, updb1, dsts1, si1, ss1))

    def zstore(i, _):
        zbuf[pl.ds(i * 16, 16)] = jnp.zeros((16,), jnp.float32)
        return 0

    lax.fori_loop(0, 200, zstore, 0)
    pltpu.sync_copy(zbuf.at[pl.ds(0, SLICE)], acc_sh.at[pl.ds(s * SLICE, SLICE)])
    pltpu.sync_copy(x_hbm, x_v)
    plsc.subcore_barrier()

    ebase, ngrp = _edge_span_32way(w)
    nch = GRP_LO // 8
    z16 = jnp.zeros((16,), jnp.float32)
    zi16 = jnp.zeros((16,), jnp.int32)

    def fire_in(k, t):
        srcb, dstb, wb, _, _, si, _ = bufs[t]
        b = ebase + k * 128
        pltpu.async_copy(src_hbm.at[pl.ds(b, 128)], srcb, si)
        pltpu.async_copy(dst_hbm.at[pl.ds(b, 128)], dstb.at[0], si)
        pltpu.async_copy(w_hbm.at[pl.ds(b, 128)], wb, si)

    def wait_in(t):
        srcb, dstb, wb, _, _, si, _ = bufs[t]
        pltpu.make_async_copy(src_hbm.at[pl.ds(0, 128)], srcb, si).wait()
        pltpu.make_async_copy(dst_hbm.at[pl.ds(0, 128)], dstb.at[0], si).wait()
        pltpu.make_async_copy(w_hbm.at[pl.ds(0, 128)], wb, si).wait()

    for t in range(2):
        srcb, dstb, wb, updb, dsts, si, ss = bufs[t]
        for g in range(8):
            updb[pl.ds(g * 16, 16)] = z16
            dsts[0, pl.ds(g * 16, 16)] = zi16
        pltpu.async_copy(updb, acc_sh.at[dsts.at[0]], ss, add=True)
        fire_in(t, t)

    def body(i, _):
        for t in range(2):
            k = 2 * i + t
            srcb, dstb, wb, updb, dsts, si, ss = bufs[t]
            wait_in(t)
            pltpu.make_async_copy(updb, acc_sh.at[dsts.at[0]], ss).wait()
            for g in range(8):
                sl = pl.ds(g * 16, 16)
                xg = plsc.load_gather(x_v, [srcb[sl]])
                updb[sl] = xg * wb[sl]
                dsts[0, sl] = dstb[0, sl]
            pltpu.async_copy(updb, acc_sh.at[dsts.at[0]], ss, add=True)

            @pl.when(k + 2 < nch - 1)
            def _():
                fire_in(k + 2, t)

        return 0

    lax.fori_loop(0, nch // 2, body, 0)
    pltpu.make_async_copy(updb0, acc_sh.at[dsts0.at[0]], ss0).wait()
    pltpu.make_async_copy(updb1, acc_sh.at[dsts1.at[0]], ss1).wait()

    def rem(k, _):
        b = ebase + (nch - 1) * 128 + k * 16
        pltpu.sync_copy(src_hbm.at[pl.ds(b, 16)], srcb16)
        pltpu.sync_copy(dst_hbm.at[pl.ds(b, 16)], dstb16.at[0])
        pltpu.sync_copy(w_hbm.at[pl.ds(b, 16)], wb16)
        xg = plsc.load_gather(x_v, [srcb16[...]])
        updb16[...] = xg * wb16[...]
        pltpu.sync_copy(updb16, acc_sh.at[dstb16.at[0]], add=True)
        return 0

    lax.fori_loop(0, 8 + ngrp - nch * 8, rem, 0)

    plsc.subcore_barrier()
    pltpu.sync_copy(acc_sh.at[pl.ds(s * SLICE, SLICE)], zbuf.at[pl.ds(0, SLICE)])
    pltpu.sync_copy(zbuf.at[pl.ds(0, SLICE)],
                    out_hbm.at[pl.ds(c * NPAD + s * SLICE, SLICE)])


@functools.partial(
    pl.kernel,
    out_type=jax.ShapeDtypeStruct((NC * NPAD,), jnp.float32),
    mesh=_mesh,
    scratch_types=(
        [pltpu.VMEM((N,), jnp.float32)]
        + [
            pltpu.VMEM((128,), jnp.int32),
            pltpu.VMEM((1, 128), jnp.int32),
            pltpu.VMEM((128,), jnp.float32),
            pltpu.VMEM((128,), jnp.float32),
            pltpu.VMEM((1, 128), jnp.int32),
        ] * 2
        + [
            pltpu.VMEM((16,), jnp.int32),
            pltpu.VMEM((1, 16), jnp.int32),
            pltpu.VMEM((16,), jnp.float32),
            pltpu.VMEM((16,), jnp.float32),
            pltpu.VMEM((3200,), jnp.float32),
        ]
        + [pltpu.SemaphoreType.DMA] * 4
        + [pltpu.VMEM_SHARED((NPAD,), jnp.float32)]
    ),
    compiler_params=_sc_params,
)
def _k1(x_hbm, src_hbm, dst_hbm, w_hbm, out_hbm, *scratch):
    _k1_body(x_hbm, src_hbm, dst_hbm, w_hbm, out_hbm, *scratch)


K2BLK = 2000


def _k2_body(aP_ref, x_ref, u_ref, v_ref, b1_ref, h1full_ref, asum_ref):
    a = aP_ref[0] + aP_ref[1]
    asum_ref[...] = a
    h = a * u_ref[...] + x_ref[...] * v_ref[...] + b1_ref[...]
    h1full_ref[...] = jnp.maximum(h, 0.0)


def _k2(aP3, x, u, v, b1r):
    grid = N // K2BLK
    return pl.pallas_call(
        _k2_body,
        grid=(grid,),
        in_specs=[
            pl.BlockSpec((NC, K2BLK, 1), lambda i: (0, i, 0)),
            pl.BlockSpec((K2BLK, 1), lambda i: (i, 0)),
            pl.BlockSpec((1, H), lambda i: (0, 0)),
            pl.BlockSpec((1, H), lambda i: (0, 0)),
            pl.BlockSpec((1, H), lambda i: (0, 0)),
        ],
        out_specs=[
            pl.BlockSpec((K2BLK, H), lambda i: (i, 0)),
            pl.BlockSpec((K2BLK, 1), lambda i: (i, 0)),
        ],
        out_shape=[
            jax.ShapeDtypeStruct((N, H), jnp.float32),
            jax.ShapeDtypeStruct((N, 1), jnp.float32),
        ],
    )(aP3, x, u, v, b1r)


GPT = G // NS
K3CH = GPT // 8
K3REM = GPT - K3CH * 8

_K3PIECES = tuple((k * 200, 200) for k in range(15)) + ((3000, 128),)


def _k3_body(a_hbm, x_hbm, u_hbm, v_hbm, b_hbm, src_hbm, dst_hbm, w_hbm,
             out_hbm,
             bnc, u_v, v_v, b_v, u_sm, v_sm, b_sm,
             srcb0, dstb0, wb0, abuf0, xbuf0, upd0, dsts0,
             srcb1, dstb1, wb1, abuf1, xbuf1, upd1, dsts1,
             dstb80, zbuf, si0, si1, sg0, sg1, ss0, ss1,
             uvb_sp, a_sp, x_sp, acc_sh):
    c = lax.axis_index("c")
    s = lax.axis_index("s")
    bufs = ((srcb0, dstb0, wb0, abuf0, xbuf0, upd0, dsts0, si0, sg0, ss0),
            (srcb1, dstb1, wb1, abuf1, xbuf1, upd1, dsts1, si1, sg1, ss1))

    def zrow(r, _):
        zbuf[r, pl.ds(0, 16)] = jnp.zeros((16,), jnp.float32)
        zbuf[r, pl.ds(16, 16)] = jnp.zeros((16,), jnp.float32)
        return 0

    lax.fori_loop(0, 200, zrow, 0)
    for off, ln in _K3PIECES:
        pltpu.sync_copy(zbuf.at[pl.ds(0, ln), :],
                        acc_sh.at[pl.ds(s * SLICE + off, ln), :])
    sl_me = pl.ds(s * SLICE, SLICE)
    bsl = pl.ds(0, SLICE)
    pltpu.sync_copy(a_hbm.at[sl_me], bnc.at[bsl])
    pltpu.sync_copy(bnc.at[bsl], a_sp.at[sl_me])
    pltpu.sync_copy(x_hbm.at[sl_me], bnc.at[bsl])
    pltpu.sync_copy(bnc.at[bsl], x_sp.at[sl_me])
    pltpu.sync_copy(u_hbm, u_v)
    pltpu.sync_copy(v_hbm, v_v)
    pltpu.sync_copy(b_hbm, b_v)
    pltpu.sync_copy(u_v, uvb_sp.at[0])
    pltpu.sync_copy(v_v, uvb_sp.at[1])
    pltpu.sync_copy(b_v, uvb_sp.at[2])
    pltpu.sync_copy(uvb_sp.at[0], u_sm)
    pltpu.sync_copy(uvb_sp.at[1], v_sm)
    pltpu.sync_copy(uvb_sp.at[2], b_sm)
    plsc.subcore_barrier()

    coff = c * 32
    u0 = u_v[pl.ds(coff, 16)]
    u1 = u_v[pl.ds(coff + 16, 16)]
    v0 = v_v[pl.ds(coff, 16)]
    v1 = v_v[pl.ds(coff + 16, 16)]
    b0 = b_v[pl.ds(coff, 16)]
    b1v = b_v[pl.ds(coff + 16, 16)]
    ebase = s * (GPT * L)
    z16 = jnp.zeros((16,), jnp.float32)
    zi16 = jnp.zeros((16,), jnp.int32)

    def fire_in(k, t):
        srcb, dstb, wb = bufs[t][0], bufs[t][1], bufs[t][2]
        si = bufs[t][7]
        b = ebase + k * 128
        pltpu.async_copy(src_hbm.at[pl.ds(b, 128)], srcb, si)
        pltpu.async_copy(dst_hbm.at[pl.ds(b, 128)], dstb.at[0], si)
        pltpu.async_copy(w_hbm.at[pl.ds(b, 128)], wb, si)

    def wait_in(t):
        srcb, dstb, wb = bufs[t][0], bufs[t][1], bufs[t][2]
        si = bufs[t][7]
        pltpu.make_async_copy(src_hbm.at[pl.ds(0, 128)], srcb, si).wait()
        pltpu.make_async_copy(dst_hbm.at[pl.ds(0, 128)], dstb.at[0], si).wait()
        pltpu.make_async_copy(w_hbm.at[pl.ds(0, 128)], wb, si).wait()

    def compute(abuf, xbuf, wb, upd, nedges):
        def go8(m, _):
            for t in range(8):
                e = m * 8 + t
                esp = jnp.full((16,), e, jnp.int32)
                asp = plsc.load_gather(abuf, [esp])
                xsp = plsc.load_gather(xbuf, [esp])
                wsp = plsc.load_gather(wb, [esp])
                h0 = jnp.maximum(asp * u0 + xsp * v0 + b0, 0.0) * wsp
                h1x = jnp.maximum(asp * u1 + xsp * v1 + b1v, 0.0) * wsp
                upd[e, pl.ds(0, 16)] = h0
                upd[e, pl.ds(16, 16)] = h1x
            return 0

        lax.fori_loop(0, nedges // 8, go8, 0)

    def fire_gathers(t):
        srcb, abuf, xbuf = bufs[t][0], bufs[t][3], bufs[t][4]
        sg = bufs[t][8]
        pltpu.async_copy(a_sp.at[srcb], abuf, sg)
        pltpu.async_copy(x_sp.at[srcb], xbuf, sg)

    def wait_gathers(t):
        srcb, abuf, xbuf = bufs[t][0], bufs[t][3], bufs[t][4]
        sg = bufs[t][8]
        pltpu.make_async_copy(a_sp.at[srcb], abuf, sg).wait()
        pltpu.make_async_copy(x_sp.at[srcb], xbuf, sg).wait()

    for t in range(2):
        srcb, dstb, wb, abuf, xbuf, upd, dsts, si, sg, ss = bufs[t]

        def zupd(r, _, upd=upd):
            upd[r, pl.ds(0, 16)] = z16
            upd[r, pl.ds(16, 16)] = z16
            return 0

        lax.fori_loop(0, 128, zupd, 0)
        for g in range(8):
            dsts[0, pl.ds(g * 16, 16)] = zi16
        pltpu.async_copy(upd, acc_sh.at[dsts.at[0]], ss, add=True)
        fire_in(t, t)

    def body(i, _):
        for t in range(2):
            k = 2 * i + t
            srcb, dstb, wb, abuf, xbuf, upd, dsts, si, sg, ss = bufs[t]
            wait_in(t)
            fire_gathers(t)
            pltpu.make_async_copy(upd, acc_sh.at[dsts.at[0]], ss).wait()
            for g in range(8):
                dsts[0, pl.ds(g * 16, 16)] = dstb[0, pl.ds(g * 16, 16)]
            wait_gathers(t)
            compute(abuf, xbuf, wb, upd, 128)
            pltpu.async_copy(upd, acc_sh.at[dsts.at[0]], ss, add=True)

            @pl.when(k + 2 < K3CH)
            def _():
                fire_in(k + 2, t)

        return 0

    lax.fori_loop(0, K3CH // 2, body, 0)
    pltpu.make_async_copy(upd0, acc_sh.at[dsts0.at[0]], ss0).wait()
    pltpu.make_async_copy(upd1, acc_sh.at[dsts1.at[0]], ss1).wait()

    b = ebase + K3CH * 128
    pltpu.sync_copy(src_hbm.at[pl.ds(b, 80)], srcb0.at[pl.ds(0, 80)])
    pltpu.sync_copy(dst_hbm.at[pl.ds(b, 80)], dstb80.at[0])
    pltpu.sync_copy(w_hbm.at[pl.ds(b, 80)], wb0.at[pl.ds(0, 80)])
    pltpu.async_copy(a_sp.at[srcb0.at[pl.ds(0, 80)]], abuf0.at[pl.ds(0, 80)],
                     sg0).wait()
    pltpu.async_copy(x_sp.at[srcb0.at[pl.ds(0, 80)]], xbuf0.at[pl.ds(0, 80)],
                     sg0).wait()
    compute(abuf0, xbuf0, wb0, upd0, 80)
    pltpu.sync_copy(upd0.at[pl.ds(0, 80), :], acc_sh.at[dstb80.at[0]], add=True)

    plsc.subcore_barrier()
    for off, ln in _K3PIECES:
        pltpu.sync_copy(acc_sh.at[pl.ds(s * SLICE + off, ln), :],
                        zbuf.at[pl.ds(0, ln), :])
        pltpu.sync_copy(zbuf.at[pl.ds(0, ln), :],
                        out_hbm.at[c, pl.ds(s * SLICE + off, ln), :])


@functools.partial(
    pl.kernel,
    out_type=jax.ShapeDtypeStruct((NC, NPAD, 32), jnp.float32),
    mesh=_mesh,
    scratch_types=(
        [
            pltpu.VMEM((SLICE,), jnp.float32),
            pltpu.VMEM((H,), jnp.float32),
            pltpu.VMEM((H,), jnp.float32),
            pltpu.VMEM((H,), jnp.float32),
            pltpu.SMEM((H,), jnp.float32),
            pltpu.SMEM((H,), jnp.float32),
            pltpu.SMEM((H,), jnp.float32),
        ]
        + [
            pltpu.VMEM((128,), jnp.int32),
            pltpu.VMEM((1, 128), jnp.int32),
            pltpu.VMEM((128,), jnp.float32),
            pltpu.VMEM((128,), jnp.float32),
            pltpu.VMEM((128,), jnp.float32),
            pltpu.VMEM((128, 32), jnp.float32),
            pltpu.VMEM((1, 128), jnp.int32),
        ] * 2
        + [
            pltpu.VMEM((1, 80), jnp.int32),
            pltpu.VMEM((200, 32), jnp.float32),
        ]
        + [pltpu.SemaphoreType.DMA] * 6
        + [
            pltpu.VMEM_SHARED((3, H), jnp.float32),
            pltpu.VMEM_SHARED((NPAD,), jnp.float32),
            pltpu.VMEM_SHARED((NPAD,), jnp.float32),
            pltpu.VMEM_SHARED((NPAD, 32), jnp.float32),
        ]
    ),
    compiler_params=_sc_params,
)
def _k3(a_hbm, x_hbm, u_hbm, v_hbm, b_hbm, src_hbm, dst_hbm, w_hbm, out_hbm,
        *scratch):
    _k3_body(a_hbm, x_hbm, u_hbm, v_hbm, b_hbm, src_hbm, dst_hbm, w_hbm,
             out_hbm, *scratch)


def _k4_body(agg_ref, h1_ref, Wrel_ref, b2_ref, Wroot_ref, h2_ref):
    a0 = agg_ref[0]
    a1 = agg_ref[1]
    Wr = Wrel_ref[...]
    dn = (((1,), (1,)), ((), ()))
    h = (lax.dot_general(a0, Wr[:, :32], dn)
         + lax.dot_general(a1, Wr[:, 32:], dn)
         + lax.dot_general(h1_ref[...], Wroot_ref[...], dn)
         + b2_ref[...])
    h2_ref[...] = jnp.maximum(h, 0.0)


def _k4(aggcat, h1full, W2_rel, b2r, W2_root):
    grid = N // K2BLK
    return pl.pallas_call(
        _k4_body,
        grid=(grid,),
        in_specs=[
            pl.BlockSpec((NC, K2BLK, 32), lambda i: (0, i, 0)),
            pl.BlockSpec((K2BLK, H), lambda i: (i, 0)),
            pl.BlockSpec((H, H), lambda i: (0, 0)),
            pl.BlockSpec((1, H), lambda i: (0, 0)),
            pl.BlockSpec((H, H), lambda i: (0, 0)),
        ],
        out_specs=pl.BlockSpec((K2BLK, H), lambda i: (i, 0)),
        out_shape=jax.ShapeDtypeStruct((N, H), jnp.float32),
    )(aggcat, h1full, W2_rel, b2r, W2_root)


def _k5_body(h2p_hbm, batch_hbm, src_hbm, dst_hbm, w_hbm, out_hbm,
             batch_v,
             srcb0, gib0, pbuf0, dstb0, wb0, rows0,
             srcb1, gib1, pbuf1, dstb1, wb1, rows1,
             gbuf, srcb16, gib16, pbuf16, dstb16, wb16, rows16,
             si0, si1, sg0, sg1, acc):
    c = lax.axis_index("c")
    s = lax.axis_index("s")
    w = _wid(c, s)
    bufs = ((srcb0, gib0, pbuf0, dstb0, wb0, rows0, si0, sg0),
            (srcb1, gib1, pbuf1, dstb1, wb1, rows1, si1, sg1))

    pltpu.sync_copy(batch_hbm, batch_v)

    def zrow(r, _):
        for c0 in range(4):
            acc[r, pl.ds(c0 * 16, 16)] = jnp.zeros((16,), jnp.float32)
        return 0

    lax.fori_loop(0, B, zrow, 0)

    ebase, ngrp = _edge_span_32way(w)
    nch = GRP_LO // 8
    iota = lax.iota(jnp.int32, 16)

    def fire_in(k, t):
        srcb, _, _, dstb, wb = bufs[t][0], None, None, bufs[t][3], bufs[t][4]
        si = bufs[t][6]
        b = ebase + k * 128
        pltpu.async_copy(src_hbm.at[pl.ds(b, 128)], srcb, si)
        pltpu.async_copy(dst_hbm.at[pl.ds(b, 128)], dstb, si)
        pltpu.async_copy(w_hbm.at[pl.ds(b, 128)], wb, si)

    def wait_in(t):
        srcb, dstb, wb = bufs[t][0], bufs[t][3], bufs[t][4]
        si = bufs[t][6]
        pltpu.make_async_copy(src_hbm.at[pl.ds(0, 128)], srcb, si).wait()
        pltpu.make_async_copy(dst_hbm.at[pl.ds(0, 128)], dstb, si).wait()
        pltpu.make_async_copy(w_hbm.at[pl.ds(0, 128)], wb, si).wait()

    def prep(t):
        srcb, gib, pbuf, rows = bufs[t][0], bufs[t][1], bufs[t][2], bufs[t][5]
        sg = bufs[t][7]
        for g in range(8):
            sl = pl.ds(g * 16, 16)
            si = srcb[sl]
            gib[sl] = lax.shift_right_logical(si, 1)
            pbuf[sl] = (si & 1) * 64
        pltpu.async_copy(h2p_hbm.at[gib], rows, sg)

    def accumulate(t):
        gib, pref, dref, wref, rows = (bufs[t][1], bufs[t][2], bufs[t][3],
                                       bufs[t][4], bufs[t][5])
        sg = bufs[t][7]
        pltpu.make_async_copy(h2p_hbm.at[gib], rows, sg).wait()
        for g in range(8):
            sl = pl.ds(g * 16, 16)
            gbuf[sl] = plsc.load_gather(batch_v, [dref[sl]])

        def acc8(m, _):
            for tt in range(8):
                e = m * 8 + tt
                esp = jnp.full((16,), e, jnp.int32)
                wsp = plsc.load_gather(wref, [esp])
                gsp = plsc.load_gather(gbuf, [esp])
                psp = plsc.load_gather(pref, [esp])
                for c0 in range(4):
                    v = plsc.load_gather(rows, [esp, psp + (iota + c0 * 16)])
                    plsc.addupdate_scatter(acc, [gsp, iota + c0 * 16], v * wsp)
            return 0

        lax.fori_loop(0, 16, acc8, 0)

    fire_in(0, 0)
    fire_in(1, 1)
    wait_in(0)
    prep(0)

    def body(i, _):
        wait_in(1)
        prep(1)
        accumulate(0)

        @pl.when(2 * i + 2 < nch)
        def _():
            fire_in(2 * i + 2, 0)

        accumulate(1)

        @pl.when(2 * i + 3 < nch)
        def _():
            fire_in(2 * i + 3, 1)

        @pl.when(2 * i + 2 < nch)
        def _():
            wait_in(0)
            prep(0)

        return 0

    lax.fori_loop(0, nch // 2, body, 0)
    accumulate(0)

    def rem(k, _):
        b = ebase + nch * 128 + k * 16
        pltpu.sync_copy(src_hbm.at[pl.ds(b, 16)], srcb16)
        pltpu.sync_copy(dst_hbm.at[pl.ds(b, 16)], dstb16)
        pltpu.sync_copy(w_hbm.at[pl.ds(b, 16)], wb16)
        si = srcb16[...]
        gib16[...] = lax.shift_right_logical(si, 1)
        pbuf16[...] = (si & 1) * 64
        pltpu.async_copy(h2p_hbm.at[gib16], rows16, sg0).wait()
        for g in range(1):
            gbuf[pl.ds(0, 16)] = plsc.load_gather(batch_v, [dstb16[...]])

        def acc1(m, _):
            for tt in range(8):
                e = m * 8 + tt
                esp = jnp.full((16,), e, jnp.int32)
                wsp = plsc.load_gather(wb16, [esp])
                gsp = plsc.load_gather(gbuf, [esp])
                psp = plsc.load_gather(pbuf16, [esp])
                for c0 in range(4):
                    v = plsc.load_gather(rows16, [esp, psp + (iota + c0 * 16)])
                    plsc.addupdate_scatter(acc, [gsp, iota + c0 * 16], v * wsp)
            return 0

        lax.fori_loop(0, 2, acc1, 0)
        return 0

    lax.fori_loop(0, ngrp - nch * 8, rem, 0)

    pltpu.sync_copy(acc, out_hbm.at[w])


@functools.partial(
    pl.kernel,
    out_type=jax.ShapeDtypeStruct((NW, B, H), jnp.float32),
    mesh=_mesh,
    scratch_types=(
        [pltpu.VMEM((N,), jnp.int32)]
        + [
            pltpu.VMEM((128,), jnp.int32),
            pltpu.VMEM((128,), jnp.int32),
            pltpu.VMEM((128,), jnp.int32),
            pltpu.VMEM((128,), jnp.int32),
            pltpu.VMEM((128,), jnp.float32),
            pltpu.VMEM((128, 128), jnp.float32),
        ] * 2
        + [
            pltpu.VMEM((128,), jnp.int32),
            pltpu.VMEM((16,), jnp.int32),
            pltpu.VMEM((16,), jnp.int32),
            pltpu.VMEM((16,), jnp.int32),
            pltpu.VMEM((16,), jnp.int32),
            pltpu.VMEM((16,), jnp.float32),
            pltpu.VMEM((16, 128), jnp.float32),
        ]
        + [pltpu.SemaphoreType.DMA] * 4
        + [pltpu.VMEM((B, H), jnp.float32)]
    ),
    compiler_params=_sc_params,
)
def _k5(h2p_hbm, batch_hbm, src_hbm, dst_hbm, w_hbm, out_hbm, *scratch):
    _k5_body(h2p_hbm, batch_hbm, src_hbm, dst_hbm, w_hbm, out_hbm, *scratch)


K6BLK = 2000


def _k6_body(h2_ref, batch_ref, parts_ref, Wrel_ref, b3_ref, Wroot_ref,
             Wlin_ref, blin_ref, out_ref, accP_ref, cnt_ref):
    i = pl.program_id(0)

    @pl.when(i == 0)
    def _():
        accP_ref[...] = jnp.zeros((B, H), jnp.float32)
        cnt_ref[...] = jnp.zeros((B, 1), jnp.float32)

    bb = batch_ref[0]
    oneT = (jnp.broadcast_to(bb, (B, K6BLK))
            == lax.broadcasted_iota(jnp.int32, (B, K6BLK), 0)
            ).astype(jnp.float32)
    dn_rowsum = (((1,), (0,)), ((), ()))
    accP_ref[...] += lax.dot_general(oneT, h2_ref[...], dn_rowsum)
    cnt_ref[...] += jnp.sum(oneT, axis=1, keepdims=True)

    @pl.when(i == (N // K6BLK) - 1)
    def _():
        Pagg = jnp.sum(parts_ref[...], axis=0)
        cnt = cnt_ref[...]
        cm = jnp.maximum(cnt, 1.0)
        dn = (((1,), (1,)), ((), ()))
        pooled = (lax.dot_general(Pagg, Wrel_ref[...], dn)
                  + cnt * b3_ref[...]
                  + lax.dot_general(accP_ref[...], Wroot_ref[...], dn)) / cm
        out_ref[...] = lax.dot_general(pooled, Wlin_ref[...], dn) + blin_ref[...]


def _k6(h2full, batch3, parts, W3_rel, b3r, W3_root, Wlin, blinr):
    grid = N // K6BLK
    return pl.pallas_call(
        _k6_body,
        grid=(grid,),
        in_specs=[
            pl.BlockSpec((K6BLK, H), lambda i: (i, 0)),
            pl.BlockSpec((1, 1, K6BLK), lambda i: (i, 0, 0)),
            pl.BlockSpec((NW, B, H), lambda i: (0, 0, 0)),
            pl.BlockSpec((H, H), lambda i: (0, 0)),
            pl.BlockSpec((1, H), lambda i: (0, 0)),
            pl.BlockSpec((H, H), lambda i: (0, 0)),
            pl.BlockSpec((OUT_C, H), lambda i: (0, 0)),
            pl.BlockSpec((1, OUT_C), lambda i: (0, 0)),
        ],
        out_specs=pl.BlockSpec((B, OUT_C), lambda i: (0, 0)),
        out_shape=jax.ShapeDtypeStruct((B, OUT_C), jnp.float32),
        scratch_shapes=[
            pltpu.VMEM((B, H), jnp.float32),
            pltpu.VMEM((B, 1), jnp.float32),
        ],
    )(h2full, batch3, parts, W3_rel, b3r, W3_root, Wlin, blinr)


def kernel(x, edge_index, batch, edge_weight,
           W1_rel, b1, W1_root, W2_rel, b2, W2_root,
           W3_rel, b3, W3_root, Wlin, blin):
    x1d = x[:, 0]
    src = edge_index[0]
    dst = edge_index[1]

    aP = _k1(x1d, src, dst, edge_weight)
    aP3 = aP.reshape(NC, NPAD, 1)

    u = W1_rel.reshape(1, H)
    v = W1_root.reshape(1, H)
    h1full, asum = _k2(aP3, x, u, v, b1.reshape(1, H))

    apad = jnp.pad(asum.reshape(N), (0, NPAD - N))
    xpad = jnp.pad(x1d, (0, NPAD - N))
    agg2cat = _k3(apad, xpad, W1_rel.reshape(H),
                  W1_root.reshape(H), b1, src, dst, edge_weight)
    h2full = _k4(agg2cat, h1full, W2_rel, b2.reshape(1, H), W2_root)

    h2pack = h2full.reshape(N // 2, 2 * H)
    parts = _k5(h2pack, batch, src, dst, edge_weight)

    batch3 = batch.reshape(N // K6BLK, 1, K6BLK)
    out = _k6(h2full, batch3, parts, W3_rel, b3.reshape(1, H),
              W3_root, Wlin, blin.reshape(1, OUT_C))
    return out


# speedup vs baseline: 1.2550x; 1.0037x over previous
import functools

import jax
import jax.numpy as jnp
from jax import lax
from jax.experimental import pallas as pl
from jax.experimental.pallas import tpu as pltpu
from jax.experimental.pallas import tpu_sc as plsc

N = 50000
E = 800000
H = 64
B = 64
OUT_C = 51

NC = 2
NS = 16
NW = NC * NS
L = 16

NPAD = 50048
SLICE = NPAD // NS
G = E // L

GRP_LO = G // NW
EXTRA = G - GRP_LO * NW

_mesh = plsc.VectorSubcoreMesh(core_axis_name="c", subcore_axis_name="s")
_sc_params = pltpu.CompilerParams(needs_layout_passes=False,
                                  use_tc_tiling_on_sc=False)


def _wid(c, s):
    return s * NC + c


def _edge_span_32way(w):
    base = w * GRP_LO + jnp.minimum(w, EXTRA)
    ngrp = GRP_LO + jnp.where(w < EXTRA, 1, 0)
    return base * L, ngrp


def _k1_body(x_hbm, src_hbm, dst_hbm, w_hbm, out_hbm,
             x_v,
             srcb0, dstb0, wb0, updb0, dsts0,
             srcb1, dstb1, wb1, updb1, dsts1,
             srcb16, dstb16, wb16, updb16,
             zbuf, si0, si1, ss0, ss1, acc_sh):
    c = lax.axis_index("c")
    s = lax.axis_index("s")
    w = _wid(c, s)
    bufs = ((srcb0, dstb0, wb0, updb0, dsts0, si0, ss0),
            (srcb1, dstb1, wb1, updb1, dsts1, si1, ss1))

    def zstore(i, _):
        zbuf[pl.ds(i * 16, 16)] = jnp.zeros((16,), jnp.float32)
        return 0

    lax.fori_loop(0, 200, zstore, 0)
    pltpu.sync_copy(zbuf.at[pl.ds(0, SLICE)], acc_sh.at[pl.ds(s * SLICE, SLICE)])
    pltpu.sync_copy(x_hbm, x_v)
    plsc.subcore_barrier()

    ebase, ngrp = _edge_span_32way(w)
    nch = GRP_LO // 8
    z16 = jnp.zeros((16,), jnp.float32)
    zi16 = jnp.zeros((16,), jnp.int32)

    def fire_in(k, t):
        srcb, dstb, wb, _, _, si, _ = bufs[t]
        b = ebase + k * 128
        pltpu.async_copy(src_hbm.at[pl.ds(b, 128)], srcb, si)
        pltpu.async_copy(dst_hbm.at[pl.ds(b, 128)], dstb.at[0], si)
        pltpu.async_copy(w_hbm.at[pl.ds(b, 128)], wb, si)

    def wait_in(t):
        srcb, dstb, wb, _, _, si, _ = bufs[t]
        pltpu.make_async_copy(src_hbm.at[pl.ds(0, 128)], srcb, si).wait()
        pltpu.make_async_copy(dst_hbm.at[pl.ds(0, 128)], dstb.at[0], si).wait()
        pltpu.make_async_copy(w_hbm.at[pl.ds(0, 128)], wb, si).wait()

    for t in range(2):
        srcb, dstb, wb, updb, dsts, si, ss = bufs[t]
        for g in range(8):
            updb[pl.ds(g * 16, 16)] = z16
            dsts[0, pl.ds(g * 16, 16)] = zi16
        pltpu.async_copy(updb, acc_sh.at[dsts.at[0]], ss, add=True)
        fire_in(t, t)

    def body(i, _):
        for t in range(2):
            k = 2 * i + t
            srcb, dstb, wb, updb, dsts, si, ss = bufs[t]
            wait_in(t)
            pltpu.make_async_copy(updb, acc_sh.at[dsts.at[0]], ss).wait()
            for g in range(8):
                sl = pl.ds(g * 16, 16)
                xg = plsc.load_gather(x_v, [srcb[sl]])
                updb[sl] = xg * wb[sl]
                dsts[0, sl] = dstb[0, sl]
            pltpu.async_copy(updb, acc_sh.at[dsts.at[0]], ss, add=True)

            @pl.when(k + 2 < nch - 1)
            def _():
                fire_in(k + 2, t)

        return 0

    lax.fori_loop(0, nch // 2, body, 0)
    pltpu.make_async_copy(updb0, acc_sh.at[dsts0.at[0]], ss0).wait()
    pltpu.make_async_copy(updb1, acc_sh.at[dsts1.at[0]], ss1).wait()

    def rem(k, _):
        b = ebase + (nch - 1) * 128 + k * 16
        pltpu.sync_copy(src_hbm.at[pl.ds(b, 16)], srcb16)
        pltpu.sync_copy(dst_hbm.at[pl.ds(b, 16)], dstb16.at[0])
        pltpu.sync_copy(w_hbm.at[pl.ds(b, 16)], wb16)
        xg = plsc.load_gather(x_v, [srcb16[...]])
        updb16[...] = xg * wb16[...]
        pltpu.sync_copy(updb16, acc_sh.at[dstb16.at[0]], add=True)
        return 0

    lax.fori_loop(0, 8 + ngrp - nch * 8, rem, 0)

    plsc.subcore_barrier()
    pltpu.sync_copy(acc_sh.at[pl.ds(s * SLICE, SLICE)], zbuf.at[pl.ds(0, SLICE)])
    pltpu.sync_copy(zbuf.at[pl.ds(0, SLICE)],
                    out_hbm.at[pl.ds(c * NPAD + s * SLICE, SLICE)])


@functools.partial(
    pl.kernel,
    out_type=jax.ShapeDtypeStruct((NC * NPAD,), jnp.float32),
    mesh=_mesh,
    scratch_types=(
        [pltpu.VMEM((N,), jnp.float32)]
        + [
            pltpu.VMEM((128,), jnp.int32),
            pltpu.VMEM((1, 128), jnp.int32),
            pltpu.VMEM((128,), jnp.float32),
            pltpu.VMEM((128,), jnp.float32),
            pltpu.VMEM((1, 128), jnp.int32),
        ] * 2
        + [
            pltpu.VMEM((16,), jnp.int32),
            pltpu.VMEM((1, 16), jnp.int32),
            pltpu.VMEM((16,), jnp.float32),
            pltpu.VMEM((16,), jnp.float32),
            pltpu.VMEM((3200,), jnp.float32),
        ]
        + [pltpu.SemaphoreType.DMA] * 4
        + [pltpu.VMEM_SHARED((NPAD,), jnp.float32)]
    ),
    compiler_params=_sc_params,
)
def _k1(x_hbm, src_hbm, dst_hbm, w_hbm, out_hbm, *scratch):
    _k1_body(x_hbm, src_hbm, dst_hbm, w_hbm, out_hbm, *scratch)


K2BLK = 2000


def _k2_body(aP_ref, x_ref, u_ref, v_ref, b1_ref, h1full_ref, asum_ref):
    a = aP_ref[0] + aP_ref[1]
    asum_ref[...] = a
    h = a * u_ref[...] + x_ref[...] * v_ref[...] + b1_ref[...]
    h1full_ref[...] = jnp.maximum(h, 0.0)


def _k2(aP3, x, u, v, b1r):
    grid = N // K2BLK
    return pl.pallas_call(
        _k2_body,
        grid=(grid,),
        in_specs=[
            pl.BlockSpec((NC, K2BLK, 1), lambda i: (0, i, 0)),
            pl.BlockSpec((K2BLK, 1), lambda i: (i, 0)),
            pl.BlockSpec((1, H), lambda i: (0, 0)),
            pl.BlockSpec((1, H), lambda i: (0, 0)),
            pl.BlockSpec((1, H), lambda i: (0, 0)),
        ],
        out_specs=[
            pl.BlockSpec((K2BLK, H), lambda i: (i, 0)),
            pl.BlockSpec((K2BLK, 1), lambda i: (i, 0)),
        ],
        out_shape=[
            jax.ShapeDtypeStruct((N, H), jnp.float32),
            jax.ShapeDtypeStruct((N, 1), jnp.float32),
        ],
    )(aP3, x, u, v, b1r)


GPT = G // NS
K3CH = GPT // 8
K3REM = GPT - K3CH * 8

_K3PIECES = tuple((k * 200, 200) for k in range(15)) + ((3000, 128),)


def _k3_body(a_hbm, x_hbm, u_hbm, v_hbm, b_hbm, src_hbm, dst_hbm, w_hbm,
             out_hbm,
             bnc, u_v, v_v, b_v, u_sm, v_sm, b_sm,
             srcb0, dstb0, wb0, abuf0, xbuf0, upd0, dsts0,
             srcb1, dstb1, wb1, abuf1, xbuf1, upd1, dsts1,
             dstb80, zbuf, si0, si1, sg0, sg1, ss0, ss1,
             uvb_sp, a_sp, x_sp, acc_sh):
    c = lax.axis_index("c")
    s = lax.axis_index("s")
    bufs = ((srcb0, dstb0, wb0, abuf0, xbuf0, upd0, dsts0, si0, sg0, ss0),
            (srcb1, dstb1, wb1, abuf1, xbuf1, upd1, dsts1, si1, sg1, ss1))

    def zrow(r, _):
        zbuf[r, pl.ds(0, 16)] = jnp.zeros((16,), jnp.float32)
        zbuf[r, pl.ds(16, 16)] = jnp.zeros((16,), jnp.float32)
        return 0

    lax.fori_loop(0, 200, zrow, 0)
    for off, ln in _K3PIECES:
        pltpu.sync_copy(zbuf.at[pl.ds(0, ln), :],
                        acc_sh.at[pl.ds(s * SLICE + off, ln), :])
    sl_me = pl.ds(s * SLICE, SLICE)
    bsl = pl.ds(0, SLICE)
    pltpu.sync_copy(a_hbm.at[sl_me], bnc.at[bsl])
    pltpu.sync_copy(bnc.at[bsl], a_sp.at[sl_me])
    pltpu.sync_copy(x_hbm.at[sl_me], bnc.at[bsl])
    pltpu.sync_copy(bnc.at[bsl], x_sp.at[sl_me])
    pltpu.sync_copy(u_hbm, u_v)
    pltpu.sync_copy(v_hbm, v_v)
    pltpu.sync_copy(b_hbm, b_v)
    pltpu.sync_copy(u_v, uvb_sp.at[0])
    pltpu.sync_copy(v_v, uvb_sp.at[1])
    pltpu.sync_copy(b_v, uvb_sp.at[2])
    pltpu.sync_copy(uvb_sp.at[0], u_sm)
    pltpu.sync_copy(uvb_sp.at[1], v_sm)
    pltpu.sync_copy(uvb_sp.at[2], b_sm)
    plsc.subcore_barrier()

    coff = c * 32
    u0 = u_v[pl.ds(coff, 16)]
    u1 = u_v[pl.ds(coff + 16, 16)]
    v0 = v_v[pl.ds(coff, 16)]
    v1 = v_v[pl.ds(coff + 16, 16)]
    b0 = b_v[pl.ds(coff, 16)]
    b1v = b_v[pl.ds(coff + 16, 16)]
    ebase = s * (GPT * L)
    z16 = jnp.zeros((16,), jnp.float32)
    zi16 = jnp.zeros((16,), jnp.int32)

    def fire_in(k, t):
        srcb, dstb, wb = bufs[t][0], bufs[t][1], bufs[t][2]
        si = bufs[t][7]
        b = ebase + k * 128
        pltpu.async_copy(src_hbm.at[pl.ds(b, 128)], srcb, si)
        pltpu.async_copy(dst_hbm.at[pl.ds(b, 128)], dstb.at[0], si)
        pltpu.async_copy(w_hbm.at[pl.ds(b, 128)], wb, si)

    def wait_in(t):
        srcb, dstb, wb = bufs[t][0], bufs[t][1], bufs[t][2]
        si = bufs[t][7]
        pltpu.make_async_copy(src_hbm.at[pl.ds(0, 128)], srcb, si).wait()
        pltpu.make_async_copy(dst_hbm.at[pl.ds(0, 128)], dstb.at[0], si).wait()
        pltpu.make_async_copy(w_hbm.at[pl.ds(0, 128)], wb, si).wait()

    def compute(abuf, xbuf, wb, upd, nedges):
        def go8(m, _):
            for t in range(16):
                e = m * 16 + t
                esp = jnp.full((16,), e, jnp.int32)
                asp = plsc.load_gather(abuf, [esp])
                xsp = plsc.load_gather(xbuf, [esp])
                wsp = plsc.load_gather(wb, [esp])
                h0 = jnp.maximum(asp * u0 + xsp * v0 + b0, 0.0) * wsp
                h1x = jnp.maximum(asp * u1 + xsp * v1 + b1v, 0.0) * wsp
                upd[e, pl.ds(0, 16)] = h0
                upd[e, pl.ds(16, 16)] = h1x
            return 0

        lax.fori_loop(0, nedges // 16, go8, 0)

    def fire_gathers(t):
        srcb, abuf, xbuf = bufs[t][0], bufs[t][3], bufs[t][4]
        sg = bufs[t][8]
        pltpu.async_copy(a_sp.at[srcb], abuf, sg)
        pltpu.async_copy(x_sp.at[srcb], xbuf, sg)

    def wait_gathers(t):
        srcb, abuf, xbuf = bufs[t][0], bufs[t][3], bufs[t][4]
        sg = bufs[t][8]
        pltpu.make_async_copy(a_sp.at[srcb], abuf, sg).wait()
        pltpu.make_async_copy(x_sp.at[srcb], xbuf, sg).wait()

    for t in range(2):
        srcb, dstb, wb, abuf, xbuf, upd, dsts, si, sg, ss = bufs[t]

        def zupd(r, _, upd=upd):
            upd[r, pl.ds(0, 16)] = z16
            upd[r, pl.ds(16, 16)] = z16
            return 0

        lax.fori_loop(0, 128, zupd, 0)
        for g in range(8):
            dsts[0, pl.ds(g * 16, 16)] = zi16
        pltpu.async_copy(upd, acc_sh.at[dsts.at[0]], ss, add=True)
        fire_in(t, t)

    def body(i, _):
        for t in range(2):
            k = 2 * i + t
            srcb, dstb, wb, abuf, xbuf, upd, dsts, si, sg, ss = bufs[t]
            wait_in(t)
            fire_gathers(t)
            pltpu.make_async_copy(upd, acc_sh.at[dsts.at[0]], ss).wait()
            for g in range(8):
                dsts[0, pl.ds(g * 16, 16)] = dstb[0, pl.ds(g * 16, 16)]
            wait_gathers(t)
            compute(abuf, xbuf, wb, upd, 128)
            pltpu.async_copy(upd, acc_sh.at[dsts.at[0]], ss, add=True)

            @pl.when(k + 2 < K3CH)
            def _():
                fire_in(k + 2, t)

        return 0

    lax.fori_loop(0, K3CH // 2, body, 0)
    pltpu.make_async_copy(upd0, acc_sh.at[dsts0.at[0]], ss0).wait()
    pltpu.make_async_copy(upd1, acc_sh.at[dsts1.at[0]], ss1).wait()

    b = ebase + K3CH * 128
    pltpu.sync_copy(src_hbm.at[pl.ds(b, 80)], srcb0.at[pl.ds(0, 80)])
    pltpu.sync_copy(dst_hbm.at[pl.ds(b, 80)], dstb80.at[0])
    pltpu.sync_copy(w_hbm.at[pl.ds(b, 80)], wb0.at[pl.ds(0, 80)])
    pltpu.async_copy(a_sp.at[srcb0.at[pl.ds(0, 80)]], abuf0.at[pl.ds(0, 80)],
                     sg0).wait()
    pltpu.async_copy(x_sp.at[srcb0.at[pl.ds(0, 80)]], xbuf0.at[pl.ds(0, 80)],
                     sg0).wait()
    compute(abuf0, xbuf0, wb0, upd0, 80)
    pltpu.sync_copy(upd0.at[pl.ds(0, 80), :], acc_sh.at[dstb80.at[0]], add=True)

    plsc.subcore_barrier()
    for off, ln in _K3PIECES:
        pltpu.sync_copy(acc_sh.at[pl.ds(s * SLICE + off, ln), :],
                        zbuf.at[pl.ds(0, ln), :])
        pltpu.sync_copy(zbuf.at[pl.ds(0, ln), :],
                        out_hbm.at[c, pl.ds(s * SLICE + off, ln), :])


@functools.partial(
    pl.kernel,
    out_type=jax.ShapeDtypeStruct((NC, NPAD, 32), jnp.float32),
    mesh=_mesh,
    scratch_types=(
        [
            pltpu.VMEM((SLICE,), jnp.float32),
            pltpu.VMEM((H,), jnp.float32),
            pltpu.VMEM((H,), jnp.float32),
            pltpu.VMEM((H,), jnp.float32),
            pltpu.SMEM((H,), jnp.float32),
            pltpu.SMEM((H,), jnp.float32),
            pltpu.SMEM((H,), jnp.float32),
        ]
        + [
            pltpu.VMEM((128,), jnp.int32),
            pltpu.VMEM((1, 128), jnp.int32),
            pltpu.VMEM((128,), jnp.float32),
            pltpu.VMEM((128,), jnp.float32),
            pltpu.VMEM((128,), jnp.float32),
            pltpu.VMEM((128, 32), jnp.float32),
            pltpu.VMEM((1, 128), jnp.int32),
        ] * 2
        + [
            pltpu.VMEM((1, 80), jnp.int32),
            pltpu.VMEM((200, 32), jnp.float32),
        ]
        + [pltpu.SemaphoreType.DMA] * 6
        + [
            pltpu.VMEM_SHARED((3, H), jnp.float32),
            pltpu.VMEM_SHARED((NPAD,), jnp.float32),
            pltpu.VMEM_SHARED((NPAD,), jnp.float32),
            pltpu.VMEM_SHARED((NPAD, 32), jnp.float32),
        ]
    ),
    compiler_params=_sc_params,
)
def _k3(a_hbm, x_hbm, u_hbm, v_hbm, b_hbm, src_hbm, dst_hbm, w_hbm, out_hbm,
        *scratch):
    _k3_body(a_hbm, x_hbm, u_hbm, v_hbm, b_hbm, src_hbm, dst_hbm, w_hbm,
             out_hbm, *scratch)


def _k4_body(agg_ref, h1_ref, Wrel_ref, b2_ref, Wroot_ref, h2_ref):
    a0 = agg_ref[0]
    a1 = agg_ref[1]
    Wr = Wrel_ref[...]
    dn = (((1,), (1,)), ((), ()))
    h = (lax.dot_general(a0, Wr[:, :32], dn)
         + lax.dot_general(a1, Wr[:, 32:], dn)
         + lax.dot_general(h1_ref[...], Wroot_ref[...], dn)
         + b2_ref[...])
    h2_ref[...] = jnp.maximum(h, 0.0)


def _k4(aggcat, h1full, W2_rel, b2r, W2_root):
    grid = N // K2BLK
    return pl.pallas_call(
        _k4_body,
        grid=(grid,),
        in_specs=[
            pl.BlockSpec((NC, K2BLK, 32), lambda i: (0, i, 0)),
            pl.BlockSpec((K2BLK, H), lambda i: (i, 0)),
            pl.BlockSpec((H, H), lambda i: (0, 0)),
            pl.BlockSpec((1, H), lambda i: (0, 0)),
            pl.BlockSpec((H, H), lambda i: (0, 0)),
        ],
        out_specs=pl.BlockSpec((K2BLK, H), lambda i: (i, 0)),
        out_shape=jax.ShapeDtypeStruct((N, H), jnp.float32),
    )(aggcat, h1full, W2_rel, b2r, W2_root)


def _k5_body(h2p_hbm, batch_hbm, src_hbm, dst_hbm, w_hbm, out_hbm,
             batch_v,
             srcb0, gib0, pbuf0, dstb0, wb0, rows0,
             srcb1, gib1, pbuf1, dstb1, wb1, rows1,
             gbuf, srcb16, gib16, pbuf16, dstb16, wb16, rows16,
             si0, si1, sg0, sg1, acc):
    c = lax.axis_index("c")
    s = lax.axis_index("s")
    w = _wid(c, s)
    bufs = ((srcb0, gib0, pbuf0, dstb0, wb0, rows0, si0, sg0),
            (srcb1, gib1, pbuf1, dstb1, wb1, rows1, si1, sg1))

    pltpu.sync_copy(batch_hbm, batch_v)

    def zrow(r, _):
        for c0 in range(4):
            acc[r, pl.ds(c0 * 16, 16)] = jnp.zeros((16,), jnp.float32)
        return 0

    lax.fori_loop(0, B, zrow, 0)

    ebase, ngrp = _edge_span_32way(w)
    nch = GRP_LO // 8
    iota = lax.iota(jnp.int32, 16)

    def fire_in(k, t):
        srcb, _, _, dstb, wb = bufs[t][0], None, None, bufs[t][3], bufs[t][4]
        si = bufs[t][6]
        b = ebase + k * 128
        pltpu.async_copy(src_hbm.at[pl.ds(b, 128)], srcb, si)
        pltpu.async_copy(dst_hbm.at[pl.ds(b, 128)], dstb, si)
        pltpu.async_copy(w_hbm.at[pl.ds(b, 128)], wb, si)

    def wait_in(t):
        srcb, dstb, wb = bufs[t][0], bufs[t][3], bufs[t][4]
        si = bufs[t][6]
        pltpu.make_async_copy(src_hbm.at[pl.ds(0, 128)], srcb, si).wait()
        pltpu.make_async_copy(dst_hbm.at[pl.ds(0, 128)], dstb, si).wait()
        pltpu.make_async_copy(w_hbm.at[pl.ds(0, 128)], wb, si).wait()

    def prep(t):
        srcb, gib, pbuf, rows = bufs[t][0], bufs[t][1], bufs[t][2], bufs[t][5]
        sg = bufs[t][7]
        for g in range(8):
            sl = pl.ds(g * 16, 16)
            si = srcb[sl]
            gib[sl] = lax.shift_right_logical(si, 1)
            pbuf[sl] = (si & 1) * 64
        pltpu.async_copy(h2p_hbm.at[gib], rows, sg)

    def accumulate(t):
        gib, pref, dref, wref, rows = (bufs[t][1], bufs[t][2], bufs[t][3],
                                       bufs[t][4], bufs[t][5])
        sg = bufs[t][7]
        pltpu.make_async_copy(h2p_hbm.at[gib], rows, sg).wait()
        for g in range(8):
            sl = pl.ds(g * 16, 16)
            gbuf[sl] = plsc.load_gather(batch_v, [dref[sl]])

        def acc8(m, _):
            for tt in range(16):
                e = m * 16 + tt
                esp = jnp.full((16,), e, jnp.int32)
                wsp = plsc.load_gather(wref, [esp])
                gsp = plsc.load_gather(gbuf, [esp])
                psp = plsc.load_gather(pref, [esp])
                for c0 in range(4):
                    v = plsc.load_gather(rows, [esp, psp + (iota + c0 * 16)])
                    plsc.addupdate_scatter(acc, [gsp, iota + c0 * 16], v * wsp)
            return 0

        lax.fori_loop(0, 8, acc8, 0)

    fire_in(0, 0)
    fire_in(1, 1)
    wait_in(0)
    prep(0)

    def body(i, _):
        wait_in(1)
        prep(1)
        accumulate(0)

        @pl.when(2 * i + 2 < nch)
        def _():
            fire_in(2 * i + 2, 0)

        accumulate(1)

        @pl.when(2 * i + 3 < nch)
        def _():
            fire_in(2 * i + 3, 1)

        @pl.when(2 * i + 2 < nch)
        def _():
            wait_in(0)
            prep(0)

        return 0

    lax.fori_loop(0, nch // 2, body, 0)
    accumulate(0)

    def rem(k, _):
        b = ebase + nch * 128 + k * 16
        pltpu.sync_copy(src_hbm.at[pl.ds(b, 16)], srcb16)
        pltpu.sync_copy(dst_hbm.at[pl.ds(b, 16)], dstb16)
        pltpu.sync_copy(w_hbm.at[pl.ds(b, 16)], wb16)
        si = srcb16[...]
        gib16[...] = lax.shift_right_logical(si, 1)
        pbuf16[...] = (si & 1) * 64
        pltpu.async_copy(h2p_hbm.at[gib16], rows16, sg0).wait()
        for g in range(1):
            gbuf[pl.ds(0, 16)] = plsc.load_gather(batch_v, [dstb16[...]])

        def acc1(m, _):
            for tt in range(8):
                e = m * 8 + tt
                esp = jnp.full((16,), e, jnp.int32)
                wsp = plsc.load_gather(wb16, [esp])
                gsp = plsc.load_gather(gbuf, [esp])
                psp = plsc.load_gather(pbuf16, [esp])
                for c0 in range(4):
                    v = plsc.load_gather(rows16, [esp, psp + (iota + c0 * 16)])
                    plsc.addupdate_scatter(acc, [gsp, iota + c0 * 16], v * wsp)
            return 0

        lax.fori_loop(0, 2, acc1, 0)
        return 0

    lax.fori_loop(0, ngrp - nch * 8, rem, 0)

    pltpu.sync_copy(acc, out_hbm.at[w])


@functools.partial(
    pl.kernel,
    out_type=jax.ShapeDtypeStruct((NW, B, H), jnp.float32),
    mesh=_mesh,
    scratch_types=(
        [pltpu.VMEM((N,), jnp.int32)]
        + [
            pltpu.VMEM((128,), jnp.int32),
            pltpu.VMEM((128,), jnp.int32),
            pltpu.VMEM((128,), jnp.int32),
            pltpu.VMEM((128,), jnp.int32),
            pltpu.VMEM((128,), jnp.float32),
            pltpu.VMEM((128, 128), jnp.float32),
        ] * 2
        + [
            pltpu.VMEM((128,), jnp.int32),
            pltpu.VMEM((16,), jnp.int32),
            pltpu.VMEM((16,), jnp.int32),
            pltpu.VMEM((16,), jnp.int32),
            pltpu.VMEM((16,), jnp.int32),
            pltpu.VMEM((16,), jnp.float32),
            pltpu.VMEM((16, 128), jnp.float32),
        ]
        + [pltpu.SemaphoreType.DMA] * 4
        + [pltpu.VMEM((B, H), jnp.float32)]
    ),
    compiler_params=_sc_params,
)
def _k5(h2p_hbm, batch_hbm, src_hbm, dst_hbm, w_hbm, out_hbm, *scratch):
    _k5_body(h2p_hbm, batch_hbm, src_hbm, dst_hbm, w_hbm, out_hbm, *scratch)


K6BLK = 2000


def _k6_body(h2_ref, batch_ref, parts_ref, Wrel_ref, b3_ref, Wroot_ref,
             Wlin_ref, blin_ref, out_ref, accP_ref, cnt_ref):
    i = pl.program_id(0)

    @pl.when(i == 0)
    def _():
        accP_ref[...] = jnp.zeros((B, H), jnp.float32)
        cnt_ref[...] = jnp.zeros((B, 1), jnp.float32)

    bb = batch_ref[0]
    oneT = (jnp.broadcast_to(bb, (B, K6BLK))
            == lax.broadcasted_iota(jnp.int32, (B, K6BLK), 0)
            ).astype(jnp.float32)
    dn_rowsum = (((1,), (0,)), ((), ()))
    accP_ref[...] += lax.dot_general(oneT, h2_ref[...], dn_rowsum)
    cnt_ref[...] += jnp.sum(oneT, axis=1, keepdims=True)

    @pl.when(i == (N // K6BLK) - 1)
    def _():
        Pagg = jnp.sum(parts_ref[...], axis=0)
        cnt = cnt_ref[...]
        cm = jnp.maximum(cnt, 1.0)
        dn = (((1,), (1,)), ((), ()))
        pooled = (lax.dot_general(Pagg, Wrel_ref[...], dn)
                  + cnt * b3_ref[...]
                  + lax.dot_general(accP_ref[...], Wroot_ref[...], dn)) / cm
        out_ref[...] = lax.dot_general(pooled, Wlin_ref[...], dn) + blin_ref[...]


def _k6(h2full, batch3, parts, W3_rel, b3r, W3_root, Wlin, blinr):
    grid = N // K6BLK
    return pl.pallas_call(
        _k6_body,
        grid=(grid,),
        in_specs=[
            pl.BlockSpec((K6BLK, H), lambda i: (i, 0)),
            pl.BlockSpec((1, 1, K6BLK), lambda i: (i, 0, 0)),
            pl.BlockSpec((NW, B, H), lambda i: (0, 0, 0)),
            pl.BlockSpec((H, H), lambda i: (0, 0)),
            pl.BlockSpec((1, H), lambda i: (0, 0)),
            pl.BlockSpec((H, H), lambda i: (0, 0)),
            pl.BlockSpec((OUT_C, H), lambda i: (0, 0)),
            pl.BlockSpec((1, OUT_C), lambda i: (0, 0)),
        ],
        out_specs=pl.BlockSpec((B, OUT_C), lambda i: (0, 0)),
        out_shape=jax.ShapeDtypeStruct((B, OUT_C), jnp.float32),
        scratch_shapes=[
            pltpu.VMEM((B, H), jnp.float32),
            pltpu.VMEM((B, 1), jnp.float32),
        ],
    )(h2full, batch3, parts, W3_rel, b3r, W3_root, Wlin, blinr)


def kernel(x, edge_index, batch, edge_weight,
           W1_rel, b1, W1_root, W2_rel, b2, W2_root,
           W3_rel, b3, W3_root, Wlin, blin):
    x1d = x[:, 0]
    src = edge_index[0]
    dst = edge_index[1]

    aP = _k1(x1d, src, dst, edge_weight)
    aP3 = aP.reshape(NC, NPAD, 1)

    u = W1_rel.reshape(1, H)
    v = W1_root.reshape(1, H)
    h1full, asum = _k2(aP3, x, u, v, b1.reshape(1, H))

    apad = jnp.pad(asum.reshape(N), (0, NPAD - N))
    xpad = jnp.pad(x1d, (0, NPAD - N))
    agg2cat = _k3(apad, xpad, W1_rel.reshape(H),
                  W1_root.reshape(H), b1, src, dst, edge_weight)
    h2full = _k4(agg2cat, h1full, W2_rel, b2.reshape(1, H), W2_root)

    h2pack = h2full.reshape(N // 2, 2 * H)
    parts = _k5(h2pack, batch, src, dst, edge_weight)

    batch3 = batch.reshape(N // K6BLK, 1, K6BLK)
    out = _k6(h2full, batch3, parts, W3_rel, b3.reshape(1, H),
              W3_root, Wlin, blin.reshape(1, OUT_C))
    return out


# speedup vs baseline: 1.2683x; 1.0106x over previous
import functools

import jax
import jax.numpy as jnp
from jax import lax
from jax.experimental import pallas as pl
from jax.experimental.pallas import tpu as pltpu
from jax.experimental.pallas import tpu_sc as plsc

N = 50000
E = 800000
H = 64
B = 64
OUT_C = 51

NC = 2
NS = 16
NW = NC * NS
L = 16

NPAD = 50048
SLICE = NPAD // NS
G = E // L

GRP_LO = G // NW
EXTRA = G - GRP_LO * NW

_mesh = plsc.VectorSubcoreMesh(core_axis_name="c", subcore_axis_name="s")
_sc_params = pltpu.CompilerParams(needs_layout_passes=False,
                                  use_tc_tiling_on_sc=False)


def _wid(c, s):
    return s * NC + c


def _edge_span_32way(w):
    base = w * GRP_LO + jnp.minimum(w, EXTRA)
    ngrp = GRP_LO + jnp.where(w < EXTRA, 1, 0)
    return base * L, ngrp


def _k1_body(x_hbm, src_hbm, dst_hbm, w_hbm, out_hbm,
             x_v,
             srcb0, dstb0, wb0, updb0, dsts0,
             srcb1, dstb1, wb1, updb1, dsts1,
             srcb16, dstb16, wb16, updb16,
             zbuf, si0, si1, ss0, ss1, acc_sh):
    c = lax.axis_index("c")
    s = lax.axis_index("s")
    w = _wid(c, s)
    bufs = ((srcb0, dstb0, wb0, updb0, dsts0, si0, ss0),
            (srcb1, dstb1, wb1, updb1, dsts1, si1, ss1))

    def zstore(i, _):
        zbuf[pl.ds(i * 16, 16)] = jnp.zeros((16,), jnp.float32)
        return 0

    lax.fori_loop(0, 200, zstore, 0)
    pltpu.sync_copy(zbuf.at[pl.ds(0, SLICE)], acc_sh.at[pl.ds(s * SLICE, SLICE)])
    pltpu.sync_copy(x_hbm, x_v)
    plsc.subcore_barrier()

    ebase, ngrp = _edge_span_32way(w)
    nch = GRP_LO // 8
    z16 = jnp.zeros((16,), jnp.float32)
    zi16 = jnp.zeros((16,), jnp.int32)

    def fire_in(k, t):
        srcb, dstb, wb, _, _, si, _ = bufs[t]
        b = ebase + k * 128
        pltpu.async_copy(src_hbm.at[pl.ds(b, 128)], srcb, si)
        pltpu.async_copy(dst_hbm.at[pl.ds(b, 128)], dstb.at[0], si)
        pltpu.async_copy(w_hbm.at[pl.ds(b, 128)], wb, si)

    def wait_in(t):
        srcb, dstb, wb, _, _, si, _ = bufs[t]
        pltpu.make_async_copy(src_hbm.at[pl.ds(0, 128)], srcb, si).wait()
        pltpu.make_async_copy(dst_hbm.at[pl.ds(0, 128)], dstb.at[0], si).wait()
        pltpu.make_async_copy(w_hbm.at[pl.ds(0, 128)], wb, si).wait()

    for t in range(2):
        srcb, dstb, wb, updb, dsts, si, ss = bufs[t]
        for g in range(8):
            updb[pl.ds(g * 16, 16)] = z16
            dsts[0, pl.ds(g * 16, 16)] = zi16
        pltpu.async_copy(updb, acc_sh.at[dsts.at[0]], ss, add=True)
        fire_in(t, t)

    def body(i, _):
        for t in range(2):
            k = 2 * i + t
            srcb, dstb, wb, updb, dsts, si, ss = bufs[t]
            wait_in(t)
            pltpu.make_async_copy(updb, acc_sh.at[dsts.at[0]], ss).wait()
            for g in range(8):
                sl = pl.ds(g * 16, 16)
                xg = plsc.load_gather(x_v, [srcb[sl]])
                updb[sl] = xg * wb[sl]
                dsts[0, sl] = dstb[0, sl]
            pltpu.async_copy(updb, acc_sh.at[dsts.at[0]], ss, add=True)

            @pl.when(k + 2 < nch - 1)
            def _():
                fire_in(k + 2, t)

        return 0

    lax.fori_loop(0, nch // 2, body, 0)
    pltpu.make_async_copy(updb0, acc_sh.at[dsts0.at[0]], ss0).wait()
    pltpu.make_async_copy(updb1, acc_sh.at[dsts1.at[0]], ss1).wait()

    def rem(k, _):
        b = ebase + (nch - 1) * 128 + k * 16
        pltpu.sync_copy(src_hbm.at[pl.ds(b, 16)], srcb16)
        pltpu.sync_copy(dst_hbm.at[pl.ds(b, 16)], dstb16.at[0])
        pltpu.sync_copy(w_hbm.at[pl.ds(b, 16)], wb16)
        xg = plsc.load_gather(x_v, [srcb16[...]])
        updb16[...] = xg * wb16[...]
        pltpu.sync_copy(updb16, acc_sh.at[dstb16.at[0]], add=True)
        return 0

    lax.fori_loop(0, 8 + ngrp - nch * 8, rem, 0)

    plsc.subcore_barrier()
    pltpu.sync_copy(acc_sh.at[pl.ds(s * SLICE, SLICE)], zbuf.at[pl.ds(0, SLICE)])
    pltpu.sync_copy(zbuf.at[pl.ds(0, SLICE)],
                    out_hbm.at[pl.ds(c * NPAD + s * SLICE, SLICE)])


@functools.partial(
    pl.kernel,
    out_type=jax.ShapeDtypeStruct((NC * NPAD,), jnp.float32),
    mesh=_mesh,
    scratch_types=(
        [pltpu.VMEM((N,), jnp.float32)]
        + [
            pltpu.VMEM((128,), jnp.int32),
            pltpu.VMEM((1, 128), jnp.int32),
            pltpu.VMEM((128,), jnp.float32),
            pltpu.VMEM((128,), jnp.float32),
            pltpu.VMEM((1, 128), jnp.int32),
        ] * 2
        + [
            pltpu.VMEM((16,), jnp.int32),
            pltpu.VMEM((1, 16), jnp.int32),
            pltpu.VMEM((16,), jnp.float32),
            pltpu.VMEM((16,), jnp.float32),
            pltpu.VMEM((3200,), jnp.float32),
        ]
        + [pltpu.SemaphoreType.DMA] * 4
        + [pltpu.VMEM_SHARED((NPAD,), jnp.float32)]
    ),
    compiler_params=_sc_params,
)
def _k1(x_hbm, src_hbm, dst_hbm, w_hbm, out_hbm, *scratch):
    _k1_body(x_hbm, src_hbm, dst_hbm, w_hbm, out_hbm, *scratch)


K2BLK = 2000


def _k2_body(aP_ref, x_ref, u_ref, v_ref, b1_ref, h1full_ref, asum_ref):
    a = aP_ref[0] + aP_ref[1]
    asum_ref[...] = a
    h = a * u_ref[...] + x_ref[...] * v_ref[...] + b1_ref[...]
    h1full_ref[...] = jnp.maximum(h, 0.0)


def _k2(aP3, x, u, v, b1r):
    grid = N // K2BLK
    return pl.pallas_call(
        _k2_body,
        grid=(grid,),
        in_specs=[
            pl.BlockSpec((NC, K2BLK, 1), lambda i: (0, i, 0)),
            pl.BlockSpec((K2BLK, 1), lambda i: (i, 0)),
            pl.BlockSpec((1, H), lambda i: (0, 0)),
            pl.BlockSpec((1, H), lambda i: (0, 0)),
            pl.BlockSpec((1, H), lambda i: (0, 0)),
        ],
        out_specs=[
            pl.BlockSpec((K2BLK, H), lambda i: (i, 0)),
            pl.BlockSpec((K2BLK, 1), lambda i: (i, 0)),
        ],
        out_shape=[
            jax.ShapeDtypeStruct((N, H), jnp.float32),
            jax.ShapeDtypeStruct((N, 1), jnp.float32),
        ],
    )(aP3, x, u, v, b1r)


GPT = G // NS
K3CH = GPT // 8
K3REM = GPT - K3CH * 8

_K3PIECES = tuple((k * 200, 200) for k in range(15)) + ((3000, 128),)


def _k3_body(a_hbm, x_hbm, u_hbm, v_hbm, b_hbm, src_hbm, dst_hbm, w_hbm,
             out_hbm,
             bnc, u_v, v_v, b_v, u_sm, v_sm, b_sm,
             srcb0, dstb0, wb0, abuf0, xbuf0, upd0, dsts0,
             srcb1, dstb1, wb1, abuf1, xbuf1, upd1, dsts1,
             dstb80, zbuf, si0, si1, sg0, sg1, ss0, ss1,
             uvb_sp, a_sp, x_sp, acc_sh):
    c = lax.axis_index("c")
    s = lax.axis_index("s")
    bufs = ((srcb0, dstb0, wb0, abuf0, xbuf0, upd0, dsts0, si0, sg0, ss0),
            (srcb1, dstb1, wb1, abuf1, xbuf1, upd1, dsts1, si1, sg1, ss1))

    def zrow(r, _):
        zbuf[r, pl.ds(0, 16)] = jnp.zeros((16,), jnp.float32)
        zbuf[r, pl.ds(16, 16)] = jnp.zeros((16,), jnp.float32)
        return 0

    lax.fori_loop(0, 200, zrow, 0)
    for off, ln in _K3PIECES:
        pltpu.sync_copy(zbuf.at[pl.ds(0, ln), :],
                        acc_sh.at[pl.ds(s * SLICE + off, ln), :])
    sl_me = pl.ds(s * SLICE, SLICE)
    bsl = pl.ds(0, SLICE)
    pltpu.sync_copy(a_hbm.at[sl_me], bnc.at[bsl])
    pltpu.sync_copy(bnc.at[bsl], a_sp.at[sl_me])
    pltpu.sync_copy(x_hbm.at[sl_me], bnc.at[bsl])
    pltpu.sync_copy(bnc.at[bsl], x_sp.at[sl_me])
    pltpu.sync_copy(u_hbm, u_v)
    pltpu.sync_copy(v_hbm, v_v)
    pltpu.sync_copy(b_hbm, b_v)
    pltpu.sync_copy(u_v, uvb_sp.at[0])
    pltpu.sync_copy(v_v, uvb_sp.at[1])
    pltpu.sync_copy(b_v, uvb_sp.at[2])
    pltpu.sync_copy(uvb_sp.at[0], u_sm)
    pltpu.sync_copy(uvb_sp.at[1], v_sm)
    pltpu.sync_copy(uvb_sp.at[2], b_sm)
    plsc.subcore_barrier()

    coff = c * 32
    u0 = u_v[pl.ds(coff, 16)]
    u1 = u_v[pl.ds(coff + 16, 16)]
    v0 = v_v[pl.ds(coff, 16)]
    v1 = v_v[pl.ds(coff + 16, 16)]
    b0 = b_v[pl.ds(coff, 16)]
    b1v = b_v[pl.ds(coff + 16, 16)]
    ebase = s * (GPT * L)
    z16 = jnp.zeros((16,), jnp.float32)
    zi16 = jnp.zeros((16,), jnp.int32)

    def fire_in(k, t):
        srcb, dstb, wb = bufs[t][0], bufs[t][1], bufs[t][2]
        si = bufs[t][7]
        b = ebase + k * 128
        pltpu.async_copy(src_hbm.at[pl.ds(b, 128)], srcb, si)
        pltpu.async_copy(dst_hbm.at[pl.ds(b, 128)], dstb.at[0], si)
        pltpu.async_copy(w_hbm.at[pl.ds(b, 128)], wb, si)

    def wait_in(t):
        srcb, dstb, wb = bufs[t][0], bufs[t][1], bufs[t][2]
        si = bufs[t][7]
        pltpu.make_async_copy(src_hbm.at[pl.ds(0, 128)], srcb, si).wait()
        pltpu.make_async_copy(dst_hbm.at[pl.ds(0, 128)], dstb.at[0], si).wait()
        pltpu.make_async_copy(w_hbm.at[pl.ds(0, 128)], wb, si).wait()

    def compute(abuf, xbuf, wb, upd, nedges):
        def go8(m, _):
            for t in range(16):
                e = m * 16 + t
                esp = jnp.full((16,), e, jnp.int32)
                asp = plsc.load_gather(abuf, [esp])
                xsp = plsc.load_gather(xbuf, [esp])
                wsp = plsc.load_gather(wb, [esp])
                h0 = jnp.maximum(asp * u0 + xsp * v0 + b0, 0.0) * wsp
                h1x = jnp.maximum(asp * u1 + xsp * v1 + b1v, 0.0) * wsp
                upd[e, pl.ds(0, 16)] = h0
                upd[e, pl.ds(16, 16)] = h1x
            return 0

        lax.fori_loop(0, nedges // 16, go8, 0)

    def fire_gathers(t):
        srcb, abuf, xbuf = bufs[t][0], bufs[t][3], bufs[t][4]
        sg = bufs[t][8]
        pltpu.async_copy(a_sp.at[srcb], abuf, sg)
        pltpu.async_copy(x_sp.at[srcb], xbuf, sg)

    def wait_gathers(t):
        srcb, abuf, xbuf = bufs[t][0], bufs[t][3], bufs[t][4]
        sg = bufs[t][8]
        pltpu.make_async_copy(a_sp.at[srcb], abuf, sg).wait()
        pltpu.make_async_copy(x_sp.at[srcb], xbuf, sg).wait()

    for t in range(2):
        srcb, dstb, wb, abuf, xbuf, upd, dsts, si, sg, ss = bufs[t]

        def zupd(r, _, upd=upd):
            upd[r, pl.ds(0, 16)] = z16
            upd[r, pl.ds(16, 16)] = z16
            return 0

        lax.fori_loop(0, 128, zupd, 0)
        for g in range(8):
            dsts[0, pl.ds(g * 16, 16)] = zi16
        pltpu.async_copy(upd, acc_sh.at[dsts.at[0]], ss, add=True)
        fire_in(t, t)

    def body(i, _):
        for t in range(2):
            k = 2 * i + t
            srcb, dstb, wb, abuf, xbuf, upd, dsts, si, sg, ss = bufs[t]
            wait_in(t)
            fire_gathers(t)
            pltpu.make_async_copy(upd, acc_sh.at[dsts.at[0]], ss).wait()
            for g in range(8):
                dsts[0, pl.ds(g * 16, 16)] = dstb[0, pl.ds(g * 16, 16)]
            wait_gathers(t)
            compute(abuf, xbuf, wb, upd, 128)
            pltpu.async_copy(upd, acc_sh.at[dsts.at[0]], ss, add=True)

            @pl.when(k + 2 < K3CH)
            def _():
                fire_in(k + 2, t)

        return 0

    lax.fori_loop(0, K3CH // 2, body, 0)
    pltpu.make_async_copy(upd0, acc_sh.at[dsts0.at[0]], ss0).wait()
    pltpu.make_async_copy(upd1, acc_sh.at[dsts1.at[0]], ss1).wait()

    b = ebase + K3CH * 128
    pltpu.sync_copy(src_hbm.at[pl.ds(b, 80)], srcb0.at[pl.ds(0, 80)])
    pltpu.sync_copy(dst_hbm.at[pl.ds(b, 80)], dstb80.at[0])
    pltpu.sync_copy(w_hbm.at[pl.ds(b, 80)], wb0.at[pl.ds(0, 80)])
    pltpu.async_copy(a_sp.at[srcb0.at[pl.ds(0, 80)]], abuf0.at[pl.ds(0, 80)],
                     sg0).wait()
    pltpu.async_copy(x_sp.at[srcb0.at[pl.ds(0, 80)]], xbuf0.at[pl.ds(0, 80)],
                     sg0).wait()
    compute(abuf0, xbuf0, wb0, upd0, 80)
    pltpu.sync_copy(upd0.at[pl.ds(0, 80), :], acc_sh.at[dstb80.at[0]], add=True)

    plsc.subcore_barrier()
    for off, ln in _K3PIECES:
        pltpu.sync_copy(acc_sh.at[pl.ds(s * SLICE + off, ln), :],
                        zbuf.at[pl.ds(0, ln), :])
        pltpu.sync_copy(zbuf.at[pl.ds(0, ln), :],
                        out_hbm.at[c, pl.ds(s * SLICE + off, ln), :])


@functools.partial(
    pl.kernel,
    out_type=jax.ShapeDtypeStruct((NC, NPAD, 32), jnp.float32),
    mesh=_mesh,
    scratch_types=(
        [
            pltpu.VMEM((SLICE,), jnp.float32),
            pltpu.VMEM((H,), jnp.float32),
            pltpu.VMEM((H,), jnp.float32),
            pltpu.VMEM((H,), jnp.float32),
            pltpu.SMEM((H,), jnp.float32),
            pltpu.SMEM((H,), jnp.float32),
            pltpu.SMEM((H,), jnp.float32),
        ]
        + [
            pltpu.VMEM((128,), jnp.int32),
            pltpu.VMEM((1, 128), jnp.int32),
            pltpu.VMEM((128,), jnp.float32),
            pltpu.VMEM((128,), jnp.float32),
            pltpu.VMEM((128,), jnp.float32),
            pltpu.VMEM((128, 32), jnp.float32),
            pltpu.VMEM((1, 128), jnp.int32),
        ] * 2
        + [
            pltpu.VMEM((1, 80), jnp.int32),
            pltpu.VMEM((200, 32), jnp.float32),
        ]
        + [pltpu.SemaphoreType.DMA] * 6
        + [
            pltpu.VMEM_SHARED((3, H), jnp.float32),
            pltpu.VMEM_SHARED((NPAD,), jnp.float32),
            pltpu.VMEM_SHARED((NPAD,), jnp.float32),
            pltpu.VMEM_SHARED((NPAD, 32), jnp.float32),
        ]
    ),
    compiler_params=_sc_params,
)
def _k3(a_hbm, x_hbm, u_hbm, v_hbm, b_hbm, src_hbm, dst_hbm, w_hbm, out_hbm,
        *scratch):
    _k3_body(a_hbm, x_hbm, u_hbm, v_hbm, b_hbm, src_hbm, dst_hbm, w_hbm,
             out_hbm, *scratch)


def _k4_body(agg_ref, h1_ref, Wrel_ref, b2_ref, Wroot_ref, h2_ref):
    a0 = agg_ref[0]
    a1 = agg_ref[1]
    Wr = Wrel_ref[...]
    dn = (((1,), (1,)), ((), ()))
    h = (lax.dot_general(a0, Wr[:, :32], dn)
         + lax.dot_general(a1, Wr[:, 32:], dn)
         + lax.dot_general(h1_ref[...], Wroot_ref[...], dn)
         + b2_ref[...])
    h2_ref[...] = jnp.maximum(h, 0.0)


def _k4(aggcat, h1full, W2_rel, b2r, W2_root):
    grid = N // K2BLK
    return pl.pallas_call(
        _k4_body,
        grid=(grid,),
        in_specs=[
            pl.BlockSpec((NC, K2BLK, 32), lambda i: (0, i, 0)),
            pl.BlockSpec((K2BLK, H), lambda i: (i, 0)),
            pl.BlockSpec((H, H), lambda i: (0, 0)),
            pl.BlockSpec((1, H), lambda i: (0, 0)),
            pl.BlockSpec((H, H), lambda i: (0, 0)),
        ],
        out_specs=pl.BlockSpec((K2BLK, H), lambda i: (i, 0)),
        out_shape=jax.ShapeDtypeStruct((N, H), jnp.float32),
    )(aggcat, h1full, W2_rel, b2r, W2_root)


def _k5_body(h2p_hbm, batch_hbm, src_hbm, dst_hbm, w_hbm, out_hbm,
             batch_v,
             srcb0, gib0, pbuf0, dstb0, wb0, rows0,
             srcb1, gib1, pbuf1, dstb1, wb1, rows1,
             gbuf, srcb16, gib16, pbuf16, dstb16, wb16, rows16,
             si0, si1, sg0, sg1, acc):
    c = lax.axis_index("c")
    s = lax.axis_index("s")
    w = _wid(c, s)
    bufs = ((srcb0, gib0, pbuf0, dstb0, wb0, rows0, si0, sg0),
            (srcb1, gib1, pbuf1, dstb1, wb1, rows1, si1, sg1))

    pltpu.sync_copy(batch_hbm, batch_v)

    def zrow(r, _):
        for c0 in range(4):
            acc[r, pl.ds(c0 * 16, 16)] = jnp.zeros((16,), jnp.float32)
        return 0

    lax.fori_loop(0, B, zrow, 0)

    ebase, ngrp = _edge_span_32way(w)
    nch = GRP_LO // 8
    iota = lax.iota(jnp.int32, 16)

    def fire_in(k, t):
        srcb, _, _, dstb, wb = bufs[t][0], None, None, bufs[t][3], bufs[t][4]
        si = bufs[t][6]
        b = ebase + k * 128
        pltpu.async_copy(src_hbm.at[pl.ds(b, 128)], srcb, si)
        pltpu.async_copy(dst_hbm.at[pl.ds(b, 128)], dstb, si)
        pltpu.async_copy(w_hbm.at[pl.ds(b, 128)], wb, si)

    def wait_in(t):
        srcb, dstb, wb = bufs[t][0], bufs[t][3], bufs[t][4]
        si = bufs[t][6]
        pltpu.make_async_copy(src_hbm.at[pl.ds(0, 128)], srcb, si).wait()
        pltpu.make_async_copy(dst_hbm.at[pl.ds(0, 128)], dstb, si).wait()
        pltpu.make_async_copy(w_hbm.at[pl.ds(0, 128)], wb, si).wait()

    def prep(t):
        srcb, gib, pbuf, rows = bufs[t][0], bufs[t][1], bufs[t][2], bufs[t][5]
        sg = bufs[t][7]
        for g in range(8):
            sl = pl.ds(g * 16, 16)
            si = srcb[sl]
            gib[sl] = lax.shift_right_logical(si, 1)
            pbuf[sl] = (si & 1) * 64
        pltpu.async_copy(h2p_hbm.at[gib], rows, sg)

    def accumulate(t):
        gib, pref, dref, wref, rows = (bufs[t][1], bufs[t][2], bufs[t][3],
                                       bufs[t][4], bufs[t][5])
        sg = bufs[t][7]
        pltpu.make_async_copy(h2p_hbm.at[gib], rows, sg).wait()
        for g in range(8):
            sl = pl.ds(g * 16, 16)
            gbuf[sl] = plsc.load_gather(batch_v, [dref[sl]])

        def acc8(m, _):
            for tt in range(16):
                e = m * 16 + tt
                esp = jnp.full((16,), e, jnp.int32)
                wsp = plsc.load_gather(wref, [esp])
                gsp = plsc.load_gather(gbuf, [esp])
                psp = plsc.load_gather(pref, [esp])
                for c0 in range(4):
                    v = plsc.load_gather(rows, [esp, psp + (iota + c0 * 16)])
                    plsc.addupdate_scatter(acc, [gsp, iota + c0 * 16], v * wsp)
            return 0

        lax.fori_loop(0, 8, acc8, 0)

    fire_in(0, 0)
    fire_in(1, 1)
    wait_in(0)
    prep(0)

    def body(i, _):
        wait_in(1)
        prep(1)
        accumulate(0)

        @pl.when(2 * i + 2 < nch)
        def _():
            fire_in(2 * i + 2, 0)

        accumulate(1)

        @pl.when(2 * i + 3 < nch)
        def _():
            fire_in(2 * i + 3, 1)

        @pl.when(2 * i + 2 < nch)
        def _():
            wait_in(0)
            prep(0)

        return 0

    lax.fori_loop(0, nch // 2, body, 0)
    accumulate(0)

    def rem(k, _):
        b = ebase + nch * 128 + k * 16
        pltpu.sync_copy(src_hbm.at[pl.ds(b, 16)], srcb16)
        pltpu.sync_copy(dst_hbm.at[pl.ds(b, 16)], dstb16)
        pltpu.sync_copy(w_hbm.at[pl.ds(b, 16)], wb16)
        si = srcb16[...]
        gib16[...] = lax.shift_right_logical(si, 1)
        pbuf16[...] = (si & 1) * 64
        pltpu.async_copy(h2p_hbm.at[gib16], rows16, sg0).wait()
        for g in range(1):
            gbuf[pl.ds(0, 16)] = plsc.load_gather(batch_v, [dstb16[...]])

        def acc1(m, _):
            for tt in range(8):
                e = m * 8 + tt
                esp = jnp.full((16,), e, jnp.int32)
                wsp = plsc.load_gather(wb16, [esp])
                gsp = plsc.load_gather(gbuf, [esp])
                psp = plsc.load_gather(pbuf16, [esp])
                for c0 in range(4):
                    v = plsc.load_gather(rows16, [esp, psp + (iota + c0 * 16)])
                    plsc.addupdate_scatter(acc, [gsp, iota + c0 * 16], v * wsp)
            return 0

        lax.fori_loop(0, 2, acc1, 0)
        return 0

    lax.fori_loop(0, ngrp - nch * 8, rem, 0)

    pltpu.sync_copy(acc, out_hbm.at[w])


@functools.partial(
    pl.kernel,
    out_type=jax.ShapeDtypeStruct((NW, B, H), jnp.float32),
    mesh=_mesh,
    scratch_types=(
        [pltpu.VMEM((N,), jnp.int32)]
        + [
            pltpu.VMEM((128,), jnp.int32),
            pltpu.VMEM((128,), jnp.int32),
            pltpu.VMEM((128,), jnp.int32),
            pltpu.VMEM((128,), jnp.int32),
            pltpu.VMEM((128,), jnp.float32),
            pltpu.VMEM((128, 128), jnp.float32),
        ] * 2
        + [
            pltpu.VMEM((128,), jnp.int32),
            pltpu.VMEM((16,), jnp.int32),
            pltpu.VMEM((16,), jnp.int32),
            pltpu.VMEM((16,), jnp.int32),
            pltpu.VMEM((16,), jnp.int32),
            pltpu.VMEM((16,), jnp.float32),
            pltpu.VMEM((16, 128), jnp.float32),
        ]
        + [pltpu.SemaphoreType.DMA] * 4
        + [pltpu.VMEM((B, H), jnp.float32)]
    ),
    compiler_params=_sc_params,
)
def _k5(h2p_hbm, batch_hbm, src_hbm, dst_hbm, w_hbm, out_hbm, *scratch):
    _k5_body(h2p_hbm, batch_hbm, src_hbm, dst_hbm, w_hbm, out_hbm, *scratch)


K6BLK = 2000


def _k6a_body(h2_ref, batch_ref, accP_ref, cnt_ref):
    i = pl.program_id(0)

    @pl.when(i == 0)
    def _():
        accP_ref[...] = jnp.zeros((B, H), jnp.float32)
        cnt_ref[...] = jnp.zeros((B, 1), jnp.float32)

    bb = batch_ref[0]
    oneT = (jnp.broadcast_to(bb, (B, K6BLK))
            == lax.broadcasted_iota(jnp.int32, (B, K6BLK), 0)
            ).astype(jnp.float32)
    dn_rowsum = (((1,), (0,)), ((), ()))
    accP_ref[...] += lax.dot_general(oneT, h2_ref[...], dn_rowsum)
    cnt_ref[...] += jnp.sum(oneT, axis=1, keepdims=True)


def _k6a(h2full, batch3):
    grid = N // K6BLK
    return pl.pallas_call(
        _k6a_body,
        grid=(grid,),
        in_specs=[
            pl.BlockSpec((K6BLK, H), lambda i: (i, 0)),
            pl.BlockSpec((1, 1, K6BLK), lambda i: (i, 0, 0)),
        ],
        out_specs=[
            pl.BlockSpec((B, H), lambda i: (0, 0)),
            pl.BlockSpec((B, 1), lambda i: (0, 0)),
        ],
        out_shape=[
            jax.ShapeDtypeStruct((B, H), jnp.float32),
            jax.ShapeDtypeStruct((B, 1), jnp.float32),
        ],
    )(h2full, batch3)


def _k6b_body(accP_ref, cnt_ref, parts_ref, Wrel_ref, b3_ref, Wroot_ref,
              Wlin_ref, blin_ref, out_ref):
    Pagg = jnp.sum(parts_ref[...], axis=0)
    cnt = cnt_ref[...]
    cm = jnp.maximum(cnt, 1.0)
    dn = (((1,), (1,)), ((), ()))
    pooled = (lax.dot_general(Pagg, Wrel_ref[...], dn)
              + cnt * b3_ref[...]
              + lax.dot_general(accP_ref[...], Wroot_ref[...], dn)) / cm
    out_ref[...] = lax.dot_general(pooled, Wlin_ref[...], dn) + blin_ref[...]


def _k6b(accP, cnt, parts, W3_rel, b3r, W3_root, Wlin, blinr):
    return pl.pallas_call(
        _k6b_body,
        out_shape=jax.ShapeDtypeStruct((B, OUT_C), jnp.float32),
    )(accP, cnt, parts, W3_rel, b3r, W3_root, Wlin, blinr)


def kernel(x, edge_index, batch, edge_weight,
           W1_rel, b1, W1_root, W2_rel, b2, W2_root,
           W3_rel, b3, W3_root, Wlin, blin):
    x1d = x[:, 0]
    src = edge_index[0]
    dst = edge_index[1]

    aP = _k1(x1d, src, dst, edge_weight)
    aP3 = aP.reshape(NC, NPAD, 1)

    u = W1_rel.reshape(1, H)
    v = W1_root.reshape(1, H)
    h1full, asum = _k2(aP3, x, u, v, b1.reshape(1, H))

    apad = jnp.pad(asum.reshape(N), (0, NPAD - N))
    xpad = jnp.pad(x1d, (0, NPAD - N))
    agg2cat = _k3(apad, xpad, W1_rel.reshape(H),
                  W1_root.reshape(H), b1, src, dst, edge_weight)
    h2full = _k4(agg2cat, h1full, W2_rel, b2.reshape(1, H), W2_root)

    h2pack = h2full.reshape(N // 2, 2 * H)
    parts = _k5(h2pack, batch, src, dst, edge_weight)

    batch3 = batch.reshape(N // K6BLK, 1, K6BLK)
    accP, cnt = _k6a(h2full, batch3)
    out = _k6b(accP, cnt, parts, W3_rel, b3.reshape(1, H),
               W3_root, Wlin, blin.reshape(1, OUT_C))
    return out
